# baseline (speedup 1.0000x reference)
import functools

import jax
import jax.numpy as jnp
from jax import lax
from jax.experimental import pallas as pl
from jax.experimental.pallas import tpu as pltpu
from jax.experimental.pallas import tpu_sc as plsc

N_NODES = 10000
E_EDGES = 320000
W = 128
EPS = 1e-5

NC = 2
NS = 16
NW = NC * NS
E_PER_W = E_EDGES // NW
CHUNK = 80
NCHUNK = E_PER_W // CHUNK
NPAD = 10240
N_PER_TILE = NPAD // NS

BE = 2000
NBE = E_EDGES // BE
BN_ROWS = 2000
NBN = N_NODES // BN_ROWS

_SELU_ALPHA = 1.6732632423543772848170429916717
_SELU_SCALE = 1.0507009873554804934193349852946


def _selu(x):
    return _SELU_SCALE * jnp.where(x > 0, x, _SELU_ALPHA * (jnp.exp(x) - 1.0))


def _dot(a, b):
    return jnp.dot(a, b, preferred_element_type=jnp.float32)


def _node_init_body(c_ref, wn_ref, bn_ref, a_ref, b_ref, be1_ref,
                    v_ref, p_ref, q_ref):
    v = _dot(c_ref[...], wn_ref[...]) + bn_ref[0:1, :]
    v_ref[...] = v
    p_ref[...] = _dot(v, a_ref[...]) + be1_ref[0:1, :]
    q_ref[...] = _dot(v, b_ref[...])


def _node_init(c, wn, bn, a, b, be1):
    out_sh = jax.ShapeDtypeStruct((N_NODES, W), jnp.float32)
    wspec = lambda sh: pl.BlockSpec(sh, lambda i: (0,) * len(sh))
    return pl.pallas_call(
        _node_init_body,
        grid=(NBN,),
        in_specs=[
            pl.BlockSpec((BN_ROWS, 4), lambda i: (i, 0)),
            wspec((4, W)), wspec((8, W)), wspec((W, W)), wspec((W, W)),
            wspec((8, W)),
        ],
        out_specs=[pl.BlockSpec((BN_ROWS, W), lambda i: (i, 0))] * 3,
        out_shape=[out_sh] * 3,
    )(c, wn, bn, a, b, be1)


def _edge0_body(gs_ref, gd_ref, er_ref, wie_ref, bie_ref, c0_ref, we2_ref,
                be2_ref, e1_ref):
    e0 = _dot(er_ref[...], wie_ref[...]) + bie_ref[0:1, :]
    h = _selu(gs_ref[...] + gd_ref[...] + _dot(e0, c0_ref[...]))
    e1_ref[...] = e0 + _dot(h, we2_ref[...]) + be2_ref[0:1, :]


def _edge0(gs, gd, e_raw, wie, bie, c0, we2, be2):
    wspec = lambda sh: pl.BlockSpec(sh, lambda i: (0,) * len(sh))
    espec = pl.BlockSpec((BE, W), lambda i: (i, 0))
    return pl.pallas_call(
        _edge0_body,
        grid=(NBE,),
        in_specs=[
            espec, espec,
            pl.BlockSpec((BE, 4), lambda i: (i, 0)),
            wspec((4, W)), wspec((8, W)), wspec((W, W)), wspec((W, W)),
            wspec((8, W)),
        ],
        out_specs=espec,
        out_shape=jax.ShapeDtypeStruct((E_EDGES, W), jnp.float32),
    )(gs, gd, e_raw, wie, bie, c0, we2, be2)


def _edge1_body(gs_ref, gd_ref, e1_ref, c1_ref, we2_ref, be2_ref,
                e2_ref, sum_ref, sq_ref):
    h = _selu(gs_ref[...] + gd_ref[...] + _dot(e1_ref[...], c1_ref[...]))
    e2 = e1_ref[...] + _dot(h, we2_ref[...]) + be2_ref[0:1, :]
    e2_ref[...] = e2

    @pl.when(pl.program_id(0) == 0)
    def _():
        sum_ref[...] = jnp.zeros_like(sum_ref)
        sq_ref[...] = jnp.zeros_like(sq_ref)

    ps = jnp.sum(e2, axis=0, keepdims=True)
    pq = jnp.sum(e2 * e2, axis=0, keepdims=True)
    sum_ref[...] += jnp.broadcast_to(ps, sum_ref.shape)
    sq_ref[...] += jnp.broadcast_to(pq, sq_ref.shape)


def _edge1(gs, gd, e1, c1, we2, be2):
    wspec = lambda sh: pl.BlockSpec(sh, lambda i: (0,) * len(sh))
    espec = pl.BlockSpec((BE, W), lambda i: (i, 0))
    return pl.pallas_call(
        _edge1_body,
        grid=(NBE,),
        in_specs=[espec, espec, espec, wspec((W, W)), wspec((W, W)),
                  wspec((8, W))],
        out_specs=[espec, wspec((8, W)), wspec((8, W))],
        out_shape=[jax.ShapeDtypeStruct((E_EDGES, W), jnp.float32),
                   jax.ShapeDtypeStruct((8, W), jnp.float32),
                   jax.ShapeDtypeStruct((8, W), jnp.float32)],
    )(gs, gd, e1, c1, we2, be2)


def _node_step_body(v_ref, s_ref, cnt_ref, n1a_ref, n1b_ref, bn1_ref,
                    wn2_ref, bn2_ref, a_ref, b_ref, be1_ref,
                    vn_ref, p_ref, q_ref):
    s = s_ref[0] + s_ref[1]
    cnt = cnt_ref[0] + cnt_ref[1]
    m = s * (1.0 / jnp.maximum(cnt[:, 0:1], 1.0))
    h = _selu(_dot(v_ref[...], n1a_ref[...]) + _dot(m, n1b_ref[...])
              + bn1_ref[0:1, :])
    vn = v_ref[...] + _dot(h, wn2_ref[...]) + bn2_ref[0:1, :]
    vn_ref[...] = vn
    p_ref[...] = _dot(vn, a_ref[...]) + be1_ref[0:1, :]
    q_ref[...] = _dot(vn, b_ref[...])


def _node_step(v, s_parts, cnt_parts, n1a, n1b, bn1, wn2, bn2, a, b, be1):
    wspec = lambda sh: pl.BlockSpec(sh, lambda i: (0,) * len(sh))
    nspec = pl.BlockSpec((BN_ROWS, W), lambda i: (i, 0))
    return pl.pallas_call(
        _node_step_body,
        grid=(NBN,),
        in_specs=[
            nspec,
            pl.BlockSpec((NC, BN_ROWS, W), lambda i: (0, i, 0)),
            pl.BlockSpec((NC, BN_ROWS, W), lambda i: (0, i, 0)),
            wspec((W, W)), wspec((W, W)), wspec((8, W)), wspec((W, W)),
            wspec((8, W)), wspec((W, W)), wspec((W, W)), wspec((8, W)),
        ],
        out_specs=[nspec] * 3,
        out_shape=[jax.ShapeDtypeStruct((N_NODES, W), jnp.float32)] * 3,
    )(v, s_parts, cnt_parts, n1a, n1b, bn1, wn2, bn2, a, b, be1)


def _node_last_body(v_ref, s_ref, cnt_ref, n1a_ref, n1b_ref, bn1_ref,
                    wn2_ref, bn2_ref, vn_ref):
    s = s_ref[0] + s_ref[1]
    cnt = cnt_ref[0] + cnt_ref[1]
    m = s * (1.0 / jnp.maximum(cnt[:, 0:1], 1.0))
    h = _selu(_dot(v_ref[...], n1a_ref[...]) + _dot(m, n1b_ref[...])
              + bn1_ref[0:1, :])
    vn_ref[...] = v_ref[...] + _dot(h, wn2_ref[...]) + bn2_ref[0:1, :]


def _node_last(v, s_parts, cnt_parts, n1a, n1b, bn1, wn2, bn2):
    wspec = lambda sh: pl.BlockSpec(sh, lambda i: (0,) * len(sh))
    nspec = pl.BlockSpec((BN_ROWS, W), lambda i: (i, 0))
    return pl.pallas_call(
        _node_last_body,
        grid=(NBN,),
        in_specs=[
            nspec,
            pl.BlockSpec((NC, BN_ROWS, W), lambda i: (0, i, 0)),
            pl.BlockSpec((NC, BN_ROWS, W), lambda i: (0, i, 0)),
            wspec((W, W)), wspec((W, W)), wspec((8, W)), wspec((W, W)),
            wspec((8, W)),
        ],
        out_specs=nspec,
        out_shape=jax.ShapeDtypeStruct((N_NODES, W), jnp.float32),
    )(v, s_parts, cnt_parts, n1a, n1b, bn1, wn2, bn2)


def _bn_small_body(x_ref, o_ref):
    x = x_ref[...]
    mu = jnp.mean(x, axis=0, keepdims=True)
    var = jnp.mean((x - mu) * (x - mu), axis=0, keepdims=True)
    o_ref[...] = (x - mu) * lax.rsqrt(var + EPS)


def _bn_small(x):
    return pl.pallas_call(
        _bn_small_body,
        out_shape=jax.ShapeDtypeStruct(x.shape, jnp.float32),
    )(x)


def _bn_apply_body(x_ref, sum_ref, sq_ref, o_ref):
    inv_n = 1.0 / E_EDGES
    mu = sum_ref[0:1, :] * inv_n
    var = sq_ref[0:1, :] * inv_n - mu * mu
    o_ref[...] = (x_ref[...] - mu) * lax.rsqrt(var + EPS)


def _bn_apply(x, ssum, ssq):
    wspec = lambda sh: pl.BlockSpec(sh, lambda i: (0,) * len(sh))
    espec = pl.BlockSpec((BE, W), lambda i: (i, 0))
    return pl.pallas_call(
        _bn_apply_body,
        grid=(NBE,),
        in_specs=[espec, wspec((8, W)), wspec((8, W))],
        out_specs=espec,
        out_shape=jax.ShapeDtypeStruct((E_EDGES, W), jnp.float32),
    )(x, ssum, ssq)


def _sc_mesh():
    return plsc.VectorSubcoreMesh(core_axis_name="c", subcore_axis_name="s")


def _sc_gather(p, q, src3, dst3):
    CH2 = 2 * CHUNK
    nsc = NCHUNK // 2
    npair = nsc // 2

    @functools.partial(
        pl.kernel,
        out_type=(jax.ShapeDtypeStruct((E_EDGES, W), jnp.float32),
                  jax.ShapeDtypeStruct((E_EDGES, W), jnp.float32)),
        mesh=_sc_mesh(),
        scratch_types=[
            pltpu.VMEM((NCHUNK, CHUNK), jnp.int32),
            pltpu.VMEM((NCHUNK, CHUNK), jnp.int32),
            pltpu.VMEM((CH2, W), jnp.float32),
            pltpu.VMEM((CH2, W), jnp.float32),
            pltpu.VMEM((CH2, W), jnp.float32),
            pltpu.VMEM((CH2, W), jnp.float32),
            pltpu.SemaphoreType.DMA,
            pltpu.SemaphoreType.DMA,
            pltpu.SemaphoreType.DMA,
            pltpu.SemaphoreType.DMA,
        ],
    )
    def k(p_hbm, q_hbm, src_hbm, dst_hbm, gs_hbm, gd_hbm,
          idx_s, idx_d, bs0, bd0, bs1, bd1, sg0, sg1, st0, st1):
        wid = lax.axis_index("s") * NC + lax.axis_index("c")
        base = wid * E_PER_W
        pltpu.sync_copy(src_hbm.at[wid], idx_s)
        pltpu.sync_copy(dst_hbm.at[wid], idx_d)
        slots = ((bs0, bd0, sg0, st0), (bs1, bd1, sg1, st1))
        lo = pl.ds(0, CHUNK)
        hi = pl.ds(CHUNK, CHUNK)

        def issue_gather(si, slot):
            bs, bd, sg, _ = slots[slot]
            c0 = 2 * si
            pltpu.async_copy(p_hbm.at[idx_s.at[c0]], bs.at[lo], sg)
            pltpu.async_copy(p_hbm.at[idx_s.at[c0 + 1]], bs.at[hi], sg)
            pltpu.async_copy(q_hbm.at[idx_d.at[c0]], bd.at[lo], sg)
            pltpu.async_copy(q_hbm.at[idx_d.at[c0 + 1]], bd.at[hi], sg)

        def drain_gather(slot):
            bs, bd, sg, _ = slots[slot]
            for buf in (bs, bd):
                pltpu.make_async_copy(p_hbm.at[idx_s.at[0]],
                                      buf.at[lo], sg).wait()
                pltpu.make_async_copy(p_hbm.at[idx_s.at[0]],
                                      buf.at[hi], sg).wait()

        def issue_store(si, slot):
            bs, bd, _, st = slots[slot]
            rows = pl.ds(base + si * CH2, CH2)
            pltpu.async_copy(bs, gs_hbm.at[rows], st)
            pltpu.async_copy(bd, gd_hbm.at[rows], st)

        def drain_store(slot):
            bs, bd, _, st = slots[slot]
            rows = pl.ds(base, CH2)
            pltpu.make_async_copy(bs, gs_hbm.at[rows], st).wait()
            pltpu.make_async_copy(bd, gd_hbm.at[rows], st).wait()

        issue_gather(0, 0)

        def pair(j, carry):
            a = 2 * j

            @pl.when(j > 0)
            def _():
                drain_store(1)

            issue_gather(a + 1, 1)
            drain_gather(0)
            issue_store(a, 0)

            drain_store(0)

            @pl.when(a + 2 < nsc)
            def _():
                issue_gather(a + 2, 0)

            drain_gather(1)
            issue_store(a + 1, 1)
            return carry

        lax.fori_loop(0, npair, pair, 0)
        drain_store(1)
        last = NCHUNK - 1
        pltpu.async_copy(p_hbm.at[idx_s.at[last]], bs0.at[lo], sg0)
        pltpu.async_copy(q_hbm.at[idx_d.at[last]], bd0.at[lo], sg0)
        pltpu.make_async_copy(p_hbm.at[idx_s.at[0]], bs0.at[lo], sg0).wait()
        pltpu.make_async_copy(p_hbm.at[idx_s.at[0]], bd0.at[lo], sg0).wait()
        rows = pl.ds(base + last * CHUNK, CHUNK)
        pltpu.async_copy(bs0.at[lo], gs_hbm.at[rows], st0)
        pltpu.async_copy(bd0.at[lo], gd_hbm.at[rows], st0)
        pltpu.make_async_copy(bs0.at[lo], gs_hbm.at[rows], st0).wait()
        pltpu.make_async_copy(bd0.at[lo], gd_hbm.at[rows], st0).wait()

    return k(p, q, src3, dst3)


def _sc_counts(dst3, zer, ones):
    @functools.partial(
        pl.kernel,
        out_type=jax.ShapeDtypeStruct((NC * NPAD, W), jnp.float32),
        mesh=_sc_mesh(),
        scratch_types=[
            pltpu.VMEM((NCHUNK, CHUNK), jnp.int32),
            pltpu.VMEM((CHUNK, W), jnp.float32),
            pltpu.VMEM_SHARED((NPAD, W), jnp.float32),
            pltpu.SemaphoreType.DMA,
        ],
    )
    def k(dst_hbm, zer_hbm, ones_hbm, cnt_hbm, idx2d, obuf, acc_cnt, scnt):
        cid = lax.axis_index("c")
        tid = lax.axis_index("s")
        wid = tid * NC + cid
        row0 = tid * N_PER_TILE
        pltpu.sync_copy(dst_hbm.at[wid], idx2d)
        pltpu.sync_copy(ones_hbm, obuf)
        pltpu.sync_copy(zer_hbm, acc_cnt.at[pl.ds(row0, N_PER_TILE)])
        plsc.subcore_barrier()

        def step(i, carry):
            @pl.when(i > 0)
            def _():
                pltpu.make_async_copy(obuf, acc_cnt.at[idx2d.at[0]],
                                      scnt).wait()

            pltpu.async_copy(obuf, acc_cnt.at[idx2d.at[i]], scnt, add=True)
            return carry

        lax.fori_loop(0, NCHUNK, step, 0)
        pltpu.make_async_copy(obuf, acc_cnt.at[idx2d.at[0]], scnt).wait()
        plsc.subcore_barrier()
        pltpu.sync_copy(acc_cnt.at[pl.ds(row0, N_PER_TILE)],
                        cnt_hbm.at[pl.ds(cid * NPAD + row0, N_PER_TILE)])

    return k(dst3, zer, ones)


def _sc_scatter(vals, dst3, zer):
    npair = NCHUNK // 2

    @functools.partial(
        pl.kernel,
        out_type=jax.ShapeDtypeStruct((NC * NPAD, W), jnp.float32),
        mesh=_sc_mesh(),
        scratch_types=[
            pltpu.VMEM((NCHUNK, CHUNK), jnp.int32),
            pltpu.VMEM((CHUNK, W), jnp.float32),
            pltpu.VMEM((CHUNK, W), jnp.float32),
            pltpu.VMEM_SHARED((NPAD, W), jnp.float32),
            pltpu.SemaphoreType.DMA,
            pltpu.SemaphoreType.DMA,
            pltpu.SemaphoreType.DMA,
            pltpu.SemaphoreType.DMA,
        ],
    )
    def k(vals_hbm, dst_hbm, zer_hbm, out_hbm,
          idx2d, vb0, vb1, acc, sl0, sl1, sc0, sc1):
        cid = lax.axis_index("c")
        tid = lax.axis_index("s")
        wid = tid * NC + cid
        base = wid * E_PER_W
        row0 = tid * N_PER_TILE
        pltpu.sync_copy(dst_hbm.at[wid], idx2d)
        pltpu.sync_copy(zer_hbm, acc.at[pl.ds(row0, N_PER_TILE)])
        plsc.subcore_barrier()
        slots = ((vb0, sl0, sc0), (vb1, sl1, sc1))

        def issue_load(ci, slot):
            vb, sl, _ = slots[slot]
            pltpu.async_copy(vals_hbm.at[pl.ds(base + ci * CHUNK, CHUNK)],
                             vb, sl)

        def drain_load(slot):
            vb, sl, _ = slots[slot]
            pltpu.make_async_copy(vals_hbm.at[pl.ds(base, CHUNK)],
                                  vb, sl).wait()

        def issue_scat(ci, slot):
            vb, _, sc = slots[slot]
            pltpu.async_copy(vb, acc.at[idx2d.at[ci]], sc, add=True)

        def drain_scat(slot):
            vb, _, sc = slots[slot]
            pltpu.make_async_copy(vb, acc.at[idx2d.at[0]], sc).wait()

        issue_load(0, 0)

        def pair(j, carry):
            a = 2 * j

            @pl.when(j > 0)
            def _():
                drain_scat(1)

            issue_load(a + 1, 1)
            drain_load(0)
            issue_scat(a, 0)

            drain_scat(0)
            issue_load(a + 2, 0)
            drain_load(1)
            issue_scat(a + 1, 1)
            return carry

        lax.fori_loop(0, npair, pair, 0)
        drain_scat(1)
        drain_load(0)
        issue_scat(NCHUNK - 1, 0)
        drain_scat(0)
        plsc.subcore_barrier()
        pltpu.sync_copy(acc.at[pl.ds(row0, N_PER_TILE)],
                        out_hbm.at[pl.ds(cid * NPAD + row0, N_PER_TILE)])

    return k(vals, dst3, zer)


def kernel(c, e, edge_index, batch, W_in_node, b_in_node, W_in_edge,
           b_in_edge, blocks):
    f32 = jnp.float32
    src3 = edge_index[0].reshape(NW, NCHUNK, CHUNK)
    dst3 = edge_index[1].reshape(NW, NCHUNK, CHUNK)

    def bc(b):
        return jnp.broadcast_to(b[None, :], (8, W)).astype(f32)

    zer = jnp.zeros((N_PER_TILE, W), f32)
    ones = jnp.ones((CHUNK, W), f32)

    p0, p1 = blocks
    a0, b0_, c0 = p0["We1"][:W], p0["We1"][W:2 * W], p0["We1"][2 * W:]
    a1, b1_, c1 = p1["We1"][:W], p1["We1"][W:2 * W], p1["We1"][2 * W:]
    n1a0, n1b0 = p0["Wn1"][:W], p0["Wn1"][W:]
    n1a1, n1b1 = p1["Wn1"][:W], p1["Wn1"][W:]

    cnt_flat = _sc_counts(dst3, zer, ones)
    cnt = cnt_flat.reshape(NC, NPAD, W)
    v0, pt0, qt0 = _node_init(c, W_in_node, bc(b_in_node), a0, b0_,
                              bc(p0["be1"]))
    gs0, gd0 = _sc_gather(pt0, qt0, src3, dst3)
    e1 = _edge0(gs0, gd0, e, W_in_edge, bc(b_in_edge), c0, p0["We2"],
                bc(p0["be2"]))
    s0_flat = _sc_scatter(e1, dst3, zer)
    s0 = s0_flat.reshape(NC, NPAD, W)
    v1, pt1, qt1 = _node_step(v0, s0, cnt, n1a0, n1b0, bc(p0["bn1"]),
                              p0["Wn2"], bc(p0["bn2"]), a1, b1_,
                              bc(p1["be1"]))
    gs1, gd1 = _sc_gather(pt1, qt1, src3, dst3)
    e2, ssum, ssq = _edge1(gs1, gd1, e1, c1, p1["We2"], bc(p1["be2"]))
    s1_flat = _sc_scatter(e2, dst3, zer)
    s1 = s1_flat.reshape(NC, NPAD, W)
    v2 = _node_last(v1, s1, cnt, n1a1, n1b1, bc(p1["bn1"]), p1["Wn2"],
                    bc(p1["bn2"]))
    c_bn = _bn_small(v2)
    e_bn = _bn_apply(e2, ssum, ssq)
    return (c_bn, e_bn, edge_index, batch)


# speedup vs baseline: 1.0062x; 1.0062x over previous
import functools

import jax
import jax.numpy as jnp
from jax import lax
from jax.experimental import pallas as pl
from jax.experimental.pallas import tpu as pltpu
from jax.experimental.pallas import tpu_sc as plsc

N_NODES = 10000
E_EDGES = 320000
W = 128
EPS = 1e-5

NC = 2
NS = 16
NW = NC * NS
E_PER_W = E_EDGES // NW
CHUNK = 80
NCHUNK = E_PER_W // CHUNK
NPAD = 10240
N_PER_TILE = NPAD // NS

BE = 2000
NBE = E_EDGES // BE
BN_ROWS = 2000
NBN = N_NODES // BN_ROWS

_SELU_ALPHA = 1.6732632423543772848170429916717
_SELU_SCALE = 1.0507009873554804934193349852946


def _selu(x):
    return _SELU_SCALE * jnp.where(x > 0, x, _SELU_ALPHA * (jnp.exp(x) - 1.0))


def _dot(a, b):
    return jnp.dot(a, b, preferred_element_type=jnp.float32)


def _node_init_body(c_ref, wn_ref, bn_ref, a_ref, b_ref, be1_ref,
                    v_ref, p_ref, q_ref):
    v = _dot(c_ref[...], wn_ref[...]) + bn_ref[0:1, :]
    v_ref[...] = v
    p_ref[...] = _dot(v, a_ref[...]) + be1_ref[0:1, :]
    q_ref[...] = _dot(v, b_ref[...])


def _node_init(c, wn, bn, a, b, be1):
    out_sh = jax.ShapeDtypeStruct((N_NODES, W), jnp.float32)
    wspec = lambda sh: pl.BlockSpec(sh, lambda i: (0,) * len(sh))
    return pl.pallas_call(
        _node_init_body,
        grid=(NBN,),
        in_specs=[
            pl.BlockSpec((BN_ROWS, 4), lambda i: (i, 0)),
            wspec((4, W)), wspec((8, W)), wspec((W, W)), wspec((W, W)),
            wspec((8, W)),
        ],
        out_specs=[pl.BlockSpec((BN_ROWS, W), lambda i: (i, 0))] * 3,
        out_shape=[out_sh] * 3,
    )(c, wn, bn, a, b, be1)


def _edge0_body(gs_ref, gd_ref, er_ref, wie_ref, bie_ref, c0_ref, we2_ref,
                be2_ref, e1_ref):
    e0 = _dot(er_ref[...], wie_ref[...]) + bie_ref[0:1, :]
    h = _selu(gs_ref[...] + gd_ref[...] + _dot(e0, c0_ref[...]))
    e1_ref[...] = e0 + _dot(h, we2_ref[...]) + be2_ref[0:1, :]


def _edge0(gs, gd, e_raw, wie, bie, c0, we2, be2):
    wspec = lambda sh: pl.BlockSpec(sh, lambda i: (0,) * len(sh))
    espec = pl.BlockSpec((BE, W), lambda i: (i, 0))
    return pl.pallas_call(
        _edge0_body,
        grid=(NBE,),
        in_specs=[
            espec, espec,
            pl.BlockSpec((BE, 4), lambda i: (i, 0)),
            wspec((4, W)), wspec((8, W)), wspec((W, W)), wspec((W, W)),
            wspec((8, W)),
        ],
        out_specs=espec,
        out_shape=jax.ShapeDtypeStruct((E_EDGES, W), jnp.float32),
    )(gs, gd, e_raw, wie, bie, c0, we2, be2)


def _edge1_body(gs_ref, gd_ref, e1_ref, c1_ref, we2_ref, be2_ref,
                e2_ref, sum_ref, sq_ref):
    h = _selu(gs_ref[...] + gd_ref[...] + _dot(e1_ref[...], c1_ref[...]))
    e2 = e1_ref[...] + _dot(h, we2_ref[...]) + be2_ref[0:1, :]
    e2_ref[...] = e2

    @pl.when(pl.program_id(0) == 0)
    def _():
        sum_ref[...] = jnp.zeros_like(sum_ref)
        sq_ref[...] = jnp.zeros_like(sq_ref)

    ps = jnp.sum(e2, axis=0, keepdims=True)
    pq = jnp.sum(e2 * e2, axis=0, keepdims=True)
    sum_ref[...] += jnp.broadcast_to(ps, sum_ref.shape)
    sq_ref[...] += jnp.broadcast_to(pq, sq_ref.shape)


def _edge1(gs, gd, e1, c1, we2, be2):
    wspec = lambda sh: pl.BlockSpec(sh, lambda i: (0,) * len(sh))
    espec = pl.BlockSpec((BE, W), lambda i: (i, 0))
    return pl.pallas_call(
        _edge1_body,
        grid=(NBE,),
        in_specs=[espec, espec, espec, wspec((W, W)), wspec((W, W)),
                  wspec((8, W))],
        out_specs=[espec, wspec((8, W)), wspec((8, W))],
        out_shape=[jax.ShapeDtypeStruct((E_EDGES, W), jnp.float32),
                   jax.ShapeDtypeStruct((8, W), jnp.float32),
                   jax.ShapeDtypeStruct((8, W), jnp.float32)],
    )(gs, gd, e1, c1, we2, be2)


def _node_step_body(v_ref, s_ref, cnt_ref, n1a_ref, n1b_ref, bn1_ref,
                    wn2_ref, bn2_ref, a_ref, b_ref, be1_ref,
                    vn_ref, p_ref, q_ref):
    s = s_ref[0] + s_ref[1]
    cnt = cnt_ref[0] + cnt_ref[1]
    m = s * (1.0 / jnp.maximum(cnt[:, 0:1], 1.0))
    h = _selu(_dot(v_ref[...], n1a_ref[...]) + _dot(m, n1b_ref[...])
              + bn1_ref[0:1, :])
    vn = v_ref[...] + _dot(h, wn2_ref[...]) + bn2_ref[0:1, :]
    vn_ref[...] = vn
    p_ref[...] = _dot(vn, a_ref[...]) + be1_ref[0:1, :]
    q_ref[...] = _dot(vn, b_ref[...])


def _node_step(v, s_parts, cnt_parts, n1a, n1b, bn1, wn2, bn2, a, b, be1):
    wspec = lambda sh: pl.BlockSpec(sh, lambda i: (0,) * len(sh))
    nspec = pl.BlockSpec((BN_ROWS, W), lambda i: (i, 0))
    return pl.pallas_call(
        _node_step_body,
        grid=(NBN,),
        in_specs=[
            nspec,
            pl.BlockSpec((NC, BN_ROWS, W), lambda i: (0, i, 0)),
            pl.BlockSpec((NC, BN_ROWS, W), lambda i: (0, i, 0)),
            wspec((W, W)), wspec((W, W)), wspec((8, W)), wspec((W, W)),
            wspec((8, W)), wspec((W, W)), wspec((W, W)), wspec((8, W)),
        ],
        out_specs=[nspec] * 3,
        out_shape=[jax.ShapeDtypeStruct((N_NODES, W), jnp.float32)] * 3,
    )(v, s_parts, cnt_parts, n1a, n1b, bn1, wn2, bn2, a, b, be1)


def _node_last_body(v_ref, s_ref, cnt_ref, n1a_ref, n1b_ref, bn1_ref,
                    wn2_ref, bn2_ref, vn_ref):
    s = s_ref[0] + s_ref[1]
    cnt = cnt_ref[0] + cnt_ref[1]
    m = s * (1.0 / jnp.maximum(cnt[:, 0:1], 1.0))
    h = _selu(_dot(v_ref[...], n1a_ref[...]) + _dot(m, n1b_ref[...])
              + bn1_ref[0:1, :])
    vn = v_ref[...] + _dot(h, wn2_ref[...]) + bn2_ref[0:1, :]
    mu = jnp.mean(vn, axis=0, keepdims=True)
    var = jnp.mean((vn - mu) * (vn - mu), axis=0, keepdims=True)
    vn_ref[...] = (vn - mu) * lax.rsqrt(var + EPS)


def _node_last(v, s_parts, cnt_parts, n1a, n1b, bn1, wn2, bn2):
    wspec = lambda sh: pl.BlockSpec(sh, lambda i: (0,) * len(sh))
    nspec = pl.BlockSpec((N_NODES, W), lambda i: (0, 0))
    return pl.pallas_call(
        _node_last_body,
        grid=(1,),
        in_specs=[
            nspec,
            pl.BlockSpec((NC, N_NODES, W), lambda i: (0, 0, 0)),
            pl.BlockSpec((NC, N_NODES, W), lambda i: (0, 0, 0)),
            wspec((W, W)), wspec((W, W)), wspec((8, W)), wspec((W, W)),
            wspec((8, W)),
        ],
        out_specs=nspec,
        out_shape=jax.ShapeDtypeStruct((N_NODES, W), jnp.float32),
    )(v, s_parts, cnt_parts, n1a, n1b, bn1, wn2, bn2)


def _bn_apply_body(x_ref, sum_ref, sq_ref, o_ref):
    inv_n = 1.0 / E_EDGES
    mu = sum_ref[0:1, :] * inv_n
    var = sq_ref[0:1, :] * inv_n - mu * mu
    o_ref[...] = (x_ref[...] - mu) * lax.rsqrt(var + EPS)


def _bn_apply(x, ssum, ssq):
    wspec = lambda sh: pl.BlockSpec(sh, lambda i: (0,) * len(sh))
    espec = pl.BlockSpec((BE, W), lambda i: (i, 0))
    return pl.pallas_call(
        _bn_apply_body,
        grid=(NBE,),
        in_specs=[espec, wspec((8, W)), wspec((8, W))],
        out_specs=espec,
        out_shape=jax.ShapeDtypeStruct((E_EDGES, W), jnp.float32),
    )(x, ssum, ssq)


def _sc_mesh():
    return plsc.VectorSubcoreMesh(core_axis_name="c", subcore_axis_name="s")


def _sc_gather(p, q, src3, dst3):
    CH2 = 2 * CHUNK
    nsc = NCHUNK // 2
    npair = nsc // 2

    @functools.partial(
        pl.kernel,
        out_type=(jax.ShapeDtypeStruct((E_EDGES, W), jnp.float32),
                  jax.ShapeDtypeStruct((E_EDGES, W), jnp.float32)),
        mesh=_sc_mesh(),
        scratch_types=[
            pltpu.VMEM((NCHUNK, CHUNK), jnp.int32),
            pltpu.VMEM((NCHUNK, CHUNK), jnp.int32),
            pltpu.VMEM((CH2, W), jnp.float32),
            pltpu.VMEM((CH2, W), jnp.float32),
            pltpu.VMEM((CH2, W), jnp.float32),
            pltpu.VMEM((CH2, W), jnp.float32),
            pltpu.SemaphoreType.DMA,
            pltpu.SemaphoreType.DMA,
            pltpu.SemaphoreType.DMA,
            pltpu.SemaphoreType.DMA,
        ],
    )
    def k(p_hbm, q_hbm, src_hbm, dst_hbm, gs_hbm, gd_hbm,
          idx_s, idx_d, bs0, bd0, bs1, bd1, sg0, sg1, st0, st1):
        wid = lax.axis_index("s") * NC + lax.axis_index("c")
        base = wid * E_PER_W
        pltpu.sync_copy(src_hbm.at[wid], idx_s)
        pltpu.sync_copy(dst_hbm.at[wid], idx_d)
        slots = ((bs0, bd0, sg0, st0), (bs1, bd1, sg1, st1))
        lo = pl.ds(0, CHUNK)
        hi = pl.ds(CHUNK, CHUNK)

        def issue_gather(si, slot):
            bs, bd, sg, _ = slots[slot]
            c0 = 2 * si
            pltpu.async_copy(p_hbm.at[idx_s.at[c0]], bs.at[lo], sg)
            pltpu.async_copy(p_hbm.at[idx_s.at[c0 + 1]], bs.at[hi], sg)
            pltpu.async_copy(q_hbm.at[idx_d.at[c0]], bd.at[lo], sg)
            pltpu.async_copy(q_hbm.at[idx_d.at[c0 + 1]], bd.at[hi], sg)

        def drain_gather(slot):
            bs, bd, sg, _ = slots[slot]
            for buf in (bs, bd):
                pltpu.make_async_copy(p_hbm.at[idx_s.at[0]],
                                      buf.at[lo], sg).wait()
                pltpu.make_async_copy(p_hbm.at[idx_s.at[0]],
                                      buf.at[hi], sg).wait()

        def issue_store(si, slot):
            bs, bd, _, st = slots[slot]
            rows = pl.ds(base + si * CH2, CH2)
            pltpu.async_copy(bs, gs_hbm.at[rows], st)
            pltpu.async_copy(bd, gd_hbm.at[rows], st)

        def drain_store(slot):
            bs, bd, _, st = slots[slot]
            rows = pl.ds(base, CH2)
            pltpu.make_async_copy(bs, gs_hbm.at[rows], st).wait()
            pltpu.make_async_copy(bd, gd_hbm.at[rows], st).wait()

        issue_gather(0, 0)

        def pair(j, carry):
            a = 2 * j

            @pl.when(j > 0)
            def _():
                drain_store(1)

            issue_gather(a + 1, 1)
            drain_gather(0)
            issue_store(a, 0)

            drain_store(0)

            @pl.when(a + 2 < nsc)
            def _():
                issue_gather(a + 2, 0)

            drain_gather(1)
            issue_store(a + 1, 1)
            return carry

        lax.fori_loop(0, npair, pair, 0)
        drain_store(1)
        last = NCHUNK - 1
        pltpu.async_copy(p_hbm.at[idx_s.at[last]], bs0.at[lo], sg0)
        pltpu.async_copy(q_hbm.at[idx_d.at[last]], bd0.at[lo], sg0)
        pltpu.make_async_copy(p_hbm.at[idx_s.at[0]], bs0.at[lo], sg0).wait()
        pltpu.make_async_copy(p_hbm.at[idx_s.at[0]], bd0.at[lo], sg0).wait()
        rows = pl.ds(base + last * CHUNK, CHUNK)
        pltpu.async_copy(bs0.at[lo], gs_hbm.at[rows], st0)
        pltpu.async_copy(bd0.at[lo], gd_hbm.at[rows], st0)
        pltpu.make_async_copy(bs0.at[lo], gs_hbm.at[rows], st0).wait()
        pltpu.make_async_copy(bd0.at[lo], gd_hbm.at[rows], st0).wait()

    return k(p, q, src3, dst3)


def _sc_counts(dst3, zer, ones):
    @functools.partial(
        pl.kernel,
        out_type=jax.ShapeDtypeStruct((NC * NPAD, W), jnp.float32),
        mesh=_sc_mesh(),
        scratch_types=[
            pltpu.VMEM((NCHUNK, CHUNK), jnp.int32),
            pltpu.VMEM((CHUNK, W), jnp.float32),
            pltpu.VMEM_SHARED((NPAD, W), jnp.float32),
            pltpu.SemaphoreType.DMA,
        ],
    )
    def k(dst_hbm, zer_hbm, ones_hbm, cnt_hbm, idx2d, obuf, acc_cnt, scnt):
        cid = lax.axis_index("c")
        tid = lax.axis_index("s")
        wid = tid * NC + cid
        row0 = tid * N_PER_TILE
        pltpu.sync_copy(dst_hbm.at[wid], idx2d)
        pltpu.sync_copy(ones_hbm, obuf)
        pltpu.sync_copy(zer_hbm, acc_cnt.at[pl.ds(row0, N_PER_TILE)])
        plsc.subcore_barrier()

        def step(i, carry):
            @pl.when(i > 0)
            def _():
                pltpu.make_async_copy(obuf, acc_cnt.at[idx2d.at[0]],
                                      scnt).wait()

            pltpu.async_copy(obuf, acc_cnt.at[idx2d.at[i]], scnt, add=True)
            return carry

        lax.fori_loop(0, NCHUNK, step, 0)
        pltpu.make_async_copy(obuf, acc_cnt.at[idx2d.at[0]], scnt).wait()
        plsc.subcore_barrier()
        pltpu.sync_copy(acc_cnt.at[pl.ds(row0, N_PER_TILE)],
                        cnt_hbm.at[pl.ds(cid * NPAD + row0, N_PER_TILE)])

    return k(dst3, zer, ones)


def _sc_scatter(vals, dst3, zer):
    npair = NCHUNK // 2

    @functools.partial(
        pl.kernel,
        out_type=jax.ShapeDtypeStruct((NC * NPAD, W), jnp.float32),
        mesh=_sc_mesh(),
        scratch_types=[
            pltpu.VMEM((NCHUNK, CHUNK), jnp.int32),
            pltpu.VMEM((CHUNK, W), jnp.float32),
            pltpu.VMEM((CHUNK, W), jnp.float32),
            pltpu.VMEM_SHARED((NPAD, W), jnp.float32),
            pltpu.SemaphoreType.DMA,
            pltpu.SemaphoreType.DMA,
            pltpu.SemaphoreType.DMA,
            pltpu.SemaphoreType.DMA,
        ],
    )
    def k(vals_hbm, dst_hbm, zer_hbm, out_hbm,
          idx2d, vb0, vb1, acc, sl0, sl1, sc0, sc1):
        cid = lax.axis_index("c")
        tid = lax.axis_index("s")
        wid = tid * NC + cid
        base = wid * E_PER_W
        row0 = tid * N_PER_TILE
        pltpu.sync_copy(dst_hbm.at[wid], idx2d)
        pltpu.sync_copy(zer_hbm, acc.at[pl.ds(row0, N_PER_TILE)])
        plsc.subcore_barrier()
        slots = ((vb0, sl0, sc0), (vb1, sl1, sc1))

        def issue_load(ci, slot):
            vb, sl, _ = slots[slot]
            pltpu.async_copy(vals_hbm.at[pl.ds(base + ci * CHUNK, CHUNK)],
                             vb, sl)

        def drain_load(slot):
            vb, sl, _ = slots[slot]
            pltpu.make_async_copy(vals_hbm.at[pl.ds(base, CHUNK)],
                                  vb, sl).wait()

        def issue_scat(ci, slot):
            vb, _, sc = slots[slot]
            pltpu.async_copy(vb, acc.at[idx2d.at[ci]], sc, add=True)

        def drain_scat(slot):
            vb, _, sc = slots[slot]
            pltpu.make_async_copy(vb, acc.at[idx2d.at[0]], sc).wait()

        issue_load(0, 0)

        def pair(j, carry):
            a = 2 * j

            @pl.when(j > 0)
            def _():
                drain_scat(1)

            issue_load(a + 1, 1)
            drain_load(0)
            issue_scat(a, 0)

            drain_scat(0)
            issue_load(a + 2, 0)
            drain_load(1)
            issue_scat(a + 1, 1)
            return carry

        lax.fori_loop(0, npair, pair, 0)
        drain_scat(1)
        drain_load(0)
        issue_scat(NCHUNK - 1, 0)
        drain_scat(0)
        plsc.subcore_barrier()
        pltpu.sync_copy(acc.at[pl.ds(row0, N_PER_TILE)],
                        out_hbm.at[pl.ds(cid * NPAD + row0, N_PER_TILE)])

    return k(vals, dst3, zer)


def kernel(c, e, edge_index, batch, W_in_node, b_in_node, W_in_edge,
           b_in_edge, blocks):
    f32 = jnp.float32
    src3 = edge_index[0].reshape(NW, NCHUNK, CHUNK)
    dst3 = edge_index[1].reshape(NW, NCHUNK, CHUNK)

    def bc(b):
        return jnp.broadcast_to(b[None, :], (8, W)).astype(f32)

    zer = jnp.zeros((N_PER_TILE, W), f32)
    ones = jnp.ones((CHUNK, W), f32)

    p0, p1 = blocks
    a0, b0_, c0 = p0["We1"][:W], p0["We1"][W:2 * W], p0["We1"][2 * W:]
    a1, b1_, c1 = p1["We1"][:W], p1["We1"][W:2 * W], p1["We1"][2 * W:]
    n1a0, n1b0 = p0["Wn1"][:W], p0["Wn1"][W:]
    n1a1, n1b1 = p1["Wn1"][:W], p1["Wn1"][W:]

    cnt_flat = _sc_counts(dst3, zer, ones)
    cnt = cnt_flat.reshape(NC, NPAD, W)
    v0, pt0, qt0 = _node_init(c, W_in_node, bc(b_in_node), a0, b0_,
                              bc(p0["be1"]))
    gs0, gd0 = _sc_gather(pt0, qt0, src3, dst3)
    e1 = _edge0(gs0, gd0, e, W_in_edge, bc(b_in_edge), c0, p0["We2"],
                bc(p0["be2"]))
    s0_flat = _sc_scatter(e1, dst3, zer)
    s0 = s0_flat.reshape(NC, NPAD, W)
    v1, pt1, qt1 = _node_step(v0, s0, cnt, n1a0, n1b0, bc(p0["bn1"]),
                              p0["Wn2"], bc(p0["bn2"]), a1, b1_,
                              bc(p1["be1"]))
    gs1, gd1 = _sc_gather(pt1, qt1, src3, dst3)
    e2, ssum, ssq = _edge1(gs1, gd1, e1, c1, p1["We2"], bc(p1["be2"]))
    s1_flat = _sc_scatter(e2, dst3, zer)
    s1 = s1_flat.reshape(NC, NPAD, W)
    c_bn = _node_last(v1, s1, cnt, n1a1, n1b1, bc(p1["bn1"]), p1["Wn2"],
                      bc(p1["bn2"]))
    e_bn = _bn_apply(e2, ssum, ssq)
    return (c_bn, e_bn, edge_index, batch)


# speedup vs baseline: 1.0866x; 1.0799x over previous
import functools

import jax
import jax.numpy as jnp
from jax import lax
from jax.experimental import pallas as pl
from jax.experimental.pallas import tpu as pltpu
from jax.experimental.pallas import tpu_sc as plsc

N_NODES = 10000
E_EDGES = 320000
W = 128
EPS = 1e-5

NC = 2
NS = 16
NW = NC * NS
E_PER_W = E_EDGES // NW
CHUNK = 80
NCHUNK = E_PER_W // CHUNK
NPAD = 10240
N_PER_TILE = NPAD // NS

BE = 4000
NBE = E_EDGES // BE
BN_ROWS = 2000
NBN = N_NODES // BN_ROWS

_SELU_ALPHA = 1.6732632423543772848170429916717
_SELU_SCALE = 1.0507009873554804934193349852946


def _selu(x):
    return _SELU_SCALE * jnp.where(x > 0, x, _SELU_ALPHA * (jnp.exp(x) - 1.0))


def _dot(a, b):
    return jnp.dot(a, b, preferred_element_type=jnp.float32)


def _node_init_body(c_ref, wn_ref, bn_ref, a_ref, b_ref, be1_ref,
                    v_ref, p_ref, q_ref):
    v = _dot(c_ref[...], wn_ref[...]) + bn_ref[0:1, :]
    v_ref[...] = v
    p_ref[...] = _dot(v, a_ref[...]) + be1_ref[0:1, :]
    q_ref[...] = _dot(v, b_ref[...])


def _node_init(c, wn, bn, a, b, be1):
    out_sh = jax.ShapeDtypeStruct((N_NODES, W), jnp.float32)
    wspec = lambda sh: pl.BlockSpec(sh, lambda i: (0,) * len(sh))
    return pl.pallas_call(
        _node_init_body,
        grid=(NBN,),
        in_specs=[
            pl.BlockSpec((BN_ROWS, 4), lambda i: (i, 0)),
            wspec((4, W)), wspec((8, W)), wspec((W, W)), wspec((W, W)),
            wspec((8, W)),
        ],
        out_specs=[pl.BlockSpec((BN_ROWS, W), lambda i: (i, 0))] * 3,
        out_shape=[out_sh] * 3,
    )(c, wn, bn, a, b, be1)


def _edge0_body(gs_ref, gd_ref, er_ref, wie_ref, bie_ref, c0_ref, we2_ref,
                be2_ref, e1_ref):
    e0 = _dot(er_ref[...], wie_ref[...]) + bie_ref[0:1, :]
    h = _selu(gs_ref[...] + gd_ref[...] + _dot(e0, c0_ref[...]))
    e1_ref[...] = e0 + _dot(h, we2_ref[...]) + be2_ref[0:1, :]


def _edge0(gs, gd, e_raw, wie, bie, c0, we2, be2):
    wspec = lambda sh: pl.BlockSpec(sh, lambda i: (0,) * len(sh))
    espec = pl.BlockSpec((BE, W), lambda i: (i, 0))
    return pl.pallas_call(
        _edge0_body,
        grid=(NBE,),
        in_specs=[
            espec, espec,
            pl.BlockSpec((BE, 4), lambda i: (i, 0)),
            wspec((4, W)), wspec((8, W)), wspec((W, W)), wspec((W, W)),
            wspec((8, W)),
        ],
        out_specs=espec,
        out_shape=jax.ShapeDtypeStruct((E_EDGES, W), jnp.float32),
    )(gs, gd, e_raw, wie, bie, c0, we2, be2)


def _edge1_body(gs_ref, gd_ref, e1_ref, c1_ref, we2_ref, be2_ref,
                e2_ref, sum_ref, sq_ref):
    h = _selu(gs_ref[...] + gd_ref[...] + _dot(e1_ref[...], c1_ref[...]))
    e2 = e1_ref[...] + _dot(h, we2_ref[...]) + be2_ref[0:1, :]
    e2_ref[...] = e2

    @pl.when(pl.program_id(0) == 0)
    def _():
        sum_ref[...] = jnp.zeros_like(sum_ref)
        sq_ref[...] = jnp.zeros_like(sq_ref)

    ps = jnp.sum(e2, axis=0, keepdims=True)
    pq = jnp.sum(e2 * e2, axis=0, keepdims=True)
    sum_ref[...] += jnp.broadcast_to(ps, sum_ref.shape)
    sq_ref[...] += jnp.broadcast_to(pq, sq_ref.shape)


def _edge1(gs, gd, e1, c1, we2, be2):
    wspec = lambda sh: pl.BlockSpec(sh, lambda i: (0,) * len(sh))
    espec = pl.BlockSpec((BE, W), lambda i: (i, 0))
    return pl.pallas_call(
        _edge1_body,
        grid=(NBE,),
        in_specs=[espec, espec, espec, wspec((W, W)), wspec((W, W)),
                  wspec((8, W))],
        out_specs=[espec, wspec((8, W)), wspec((8, W))],
        out_shape=[jax.ShapeDtypeStruct((E_EDGES, W), jnp.float32),
                   jax.ShapeDtypeStruct((8, W), jnp.float32),
                   jax.ShapeDtypeStruct((8, W), jnp.float32)],
    )(gs, gd, e1, c1, we2, be2)


def _node_step_body(v_ref, s_ref, cnt_ref, n1a_ref, n1b_ref, bn1_ref,
                    wn2_ref, bn2_ref, a_ref, b_ref, be1_ref,
                    vn_ref, p_ref, q_ref):
    s = s_ref[0] + s_ref[1]
    cnt = cnt_ref[0] + cnt_ref[1]
    m = s * (1.0 / jnp.maximum(cnt[:, 0:1], 1.0))
    h = _selu(_dot(v_ref[...], n1a_ref[...]) + _dot(m, n1b_ref[...])
              + bn1_ref[0:1, :])
    vn = v_ref[...] + _dot(h, wn2_ref[...]) + bn2_ref[0:1, :]
    vn_ref[...] = vn
    p_ref[...] = _dot(vn, a_ref[...]) + be1_ref[0:1, :]
    q_ref[...] = _dot(vn, b_ref[...])


def _node_step(v, s_parts, cnt_parts, n1a, n1b, bn1, wn2, bn2, a, b, be1):
    wspec = lambda sh: pl.BlockSpec(sh, lambda i: (0,) * len(sh))
    nspec = pl.BlockSpec((BN_ROWS, W), lambda i: (i, 0))
    return pl.pallas_call(
        _node_step_body,
        grid=(NBN,),
        in_specs=[
            nspec,
            pl.BlockSpec((NC, BN_ROWS, W), lambda i: (0, i, 0)),
            pl.BlockSpec((NC, BN_ROWS, W), lambda i: (0, i, 0)),
            wspec((W, W)), wspec((W, W)), wspec((8, W)), wspec((W, W)),
            wspec((8, W)), wspec((W, W)), wspec((W, W)), wspec((8, W)),
        ],
        out_specs=[nspec] * 3,
        out_shape=[jax.ShapeDtypeStruct((N_NODES, W), jnp.float32)] * 3,
    )(v, s_parts, cnt_parts, n1a, n1b, bn1, wn2, bn2, a, b, be1)


def _node_last_body(v_ref, s_ref, cnt_ref, n1a_ref, n1b_ref, bn1_ref,
                    wn2_ref, bn2_ref, vn_ref):
    s = s_ref[0] + s_ref[1]
    cnt = cnt_ref[0] + cnt_ref[1]
    m = s * (1.0 / jnp.maximum(cnt[:, 0:1], 1.0))
    h = _selu(_dot(v_ref[...], n1a_ref[...]) + _dot(m, n1b_ref[...])
              + bn1_ref[0:1, :])
    vn = v_ref[...] + _dot(h, wn2_ref[...]) + bn2_ref[0:1, :]
    mu = jnp.mean(vn, axis=0, keepdims=True)
    var = jnp.mean((vn - mu) * (vn - mu), axis=0, keepdims=True)
    vn_ref[...] = (vn - mu) * lax.rsqrt(var + EPS)


def _node_last(v, s_parts, cnt_parts, n1a, n1b, bn1, wn2, bn2):
    wspec = lambda sh: pl.BlockSpec(sh, lambda i: (0,) * len(sh))
    nspec = pl.BlockSpec((N_NODES, W), lambda i: (0, 0))
    return pl.pallas_call(
        _node_last_body,
        grid=(1,),
        in_specs=[
            nspec,
            pl.BlockSpec((NC, N_NODES, W), lambda i: (0, 0, 0)),
            pl.BlockSpec((NC, N_NODES, W), lambda i: (0, 0, 0)),
            wspec((W, W)), wspec((W, W)), wspec((8, W)), wspec((W, W)),
            wspec((8, W)),
        ],
        out_specs=nspec,
        out_shape=jax.ShapeDtypeStruct((N_NODES, W), jnp.float32),
    )(v, s_parts, cnt_parts, n1a, n1b, bn1, wn2, bn2)


def _bn_apply_body(x_ref, sum_ref, sq_ref, o_ref):
    inv_n = 1.0 / E_EDGES
    mu = sum_ref[0:1, :] * inv_n
    var = sq_ref[0:1, :] * inv_n - mu * mu
    o_ref[...] = (x_ref[...] - mu) * lax.rsqrt(var + EPS)


def _bn_apply(x, ssum, ssq):
    wspec = lambda sh: pl.BlockSpec(sh, lambda i: (0,) * len(sh))
    espec = pl.BlockSpec((BE, W), lambda i: (i, 0))
    return pl.pallas_call(
        _bn_apply_body,
        grid=(NBE,),
        in_specs=[espec, wspec((8, W)), wspec((8, W))],
        out_specs=espec,
        out_shape=jax.ShapeDtypeStruct((E_EDGES, W), jnp.float32),
    )(x, ssum, ssq)


def _sc_mesh():
    return plsc.VectorSubcoreMesh(core_axis_name="c", subcore_axis_name="s")


def _sc_gather(p, q, src3, dst3):
    CH2 = 2 * CHUNK
    nsc = NCHUNK // 2
    npair = nsc // 2

    @functools.partial(
        pl.kernel,
        out_type=(jax.ShapeDtypeStruct((E_EDGES, W), jnp.float32),
                  jax.ShapeDtypeStruct((E_EDGES, W), jnp.float32)),
        mesh=_sc_mesh(),
        scratch_types=[
            pltpu.VMEM((NCHUNK, CHUNK), jnp.int32),
            pltpu.VMEM((NCHUNK, CHUNK), jnp.int32),
            pltpu.VMEM((CH2, W), jnp.float32),
            pltpu.VMEM((CH2, W), jnp.float32),
            pltpu.VMEM((CH2, W), jnp.float32),
            pltpu.VMEM((CH2, W), jnp.float32),
            pltpu.SemaphoreType.DMA,
            pltpu.SemaphoreType.DMA,
            pltpu.SemaphoreType.DMA,
            pltpu.SemaphoreType.DMA,
        ],
    )
    def k(p_hbm, q_hbm, src_hbm, dst_hbm, gs_hbm, gd_hbm,
          idx_s, idx_d, bs0, bd0, bs1, bd1, sg0, sg1, st0, st1):
        wid = lax.axis_index("s") * NC + lax.axis_index("c")
        base = wid * E_PER_W
        pltpu.sync_copy(src_hbm.at[wid], idx_s)
        pltpu.sync_copy(dst_hbm.at[wid], idx_d)
        slots = ((bs0, bd0, sg0, st0), (bs1, bd1, sg1, st1))
        lo = pl.ds(0, CHUNK)
        hi = pl.ds(CHUNK, CHUNK)

        def issue_gather(si, slot):
            bs, bd, sg, _ = slots[slot]
            c0 = 2 * si
            pltpu.async_copy(p_hbm.at[idx_s.at[c0]], bs.at[lo], sg)
            pltpu.async_copy(p_hbm.at[idx_s.at[c0 + 1]], bs.at[hi], sg)
            pltpu.async_copy(q_hbm.at[idx_d.at[c0]], bd.at[lo], sg)
            pltpu.async_copy(q_hbm.at[idx_d.at[c0 + 1]], bd.at[hi], sg)

        def drain_gather(slot):
            bs, bd, sg, _ = slots[slot]
            for buf in (bs, bd):
                pltpu.make_async_copy(p_hbm.at[idx_s.at[0]],
                                      buf.at[lo], sg).wait()
                pltpu.make_async_copy(p_hbm.at[idx_s.at[0]],
                                      buf.at[hi], sg).wait()

        def issue_store(si, slot):
            bs, bd, _, st = slots[slot]
            rows = pl.ds(base + si * CH2, CH2)
            pltpu.async_copy(bs, gs_hbm.at[rows], st)
            pltpu.async_copy(bd, gd_hbm.at[rows], st)

        def drain_store(slot):
            bs, bd, _, st = slots[slot]
            rows = pl.ds(base, CH2)
            pltpu.make_async_copy(bs, gs_hbm.at[rows], st).wait()
            pltpu.make_async_copy(bd, gd_hbm.at[rows], st).wait()

        issue_gather(0, 0)

        def pair(j, carry):
            a = 2 * j

            @pl.when(j > 0)
            def _():
                drain_store(1)

            issue_gather(a + 1, 1)
            drain_gather(0)
            issue_store(a, 0)

            drain_store(0)

            @pl.when(a + 2 < nsc)
            def _():
                issue_gather(a + 2, 0)

            drain_gather(1)
            issue_store(a + 1, 1)
            return carry

        lax.fori_loop(0, npair, pair, 0)
        drain_store(1)
        last = NCHUNK - 1
        pltpu.async_copy(p_hbm.at[idx_s.at[last]], bs0.at[lo], sg0)
        pltpu.async_copy(q_hbm.at[idx_d.at[last]], bd0.at[lo], sg0)
        pltpu.make_async_copy(p_hbm.at[idx_s.at[0]], bs0.at[lo], sg0).wait()
        pltpu.make_async_copy(p_hbm.at[idx_s.at[0]], bd0.at[lo], sg0).wait()
        rows = pl.ds(base + last * CHUNK, CHUNK)
        pltpu.async_copy(bs0.at[lo], gs_hbm.at[rows], st0)
        pltpu.async_copy(bd0.at[lo], gd_hbm.at[rows], st0)
        pltpu.make_async_copy(bs0.at[lo], gs_hbm.at[rows], st0).wait()
        pltpu.make_async_copy(bd0.at[lo], gd_hbm.at[rows], st0).wait()

    return k(p, q, src3, dst3)


def _sc_counts(dst3, zer, ones):
    @functools.partial(
        pl.kernel,
        out_type=jax.ShapeDtypeStruct((NC * NPAD, W), jnp.float32),
        mesh=_sc_mesh(),
        scratch_types=[
            pltpu.VMEM((NCHUNK, CHUNK), jnp.int32),
            pltpu.VMEM((CHUNK, W), jnp.float32),
            pltpu.VMEM_SHARED((NPAD, W), jnp.float32),
            pltpu.SemaphoreType.DMA,
        ],
    )
    def k(dst_hbm, zer_hbm, ones_hbm, cnt_hbm, idx2d, obuf, acc_cnt, scnt):
        cid = lax.axis_index("c")
        tid = lax.axis_index("s")
        wid = tid * NC + cid
        row0 = tid * N_PER_TILE
        pltpu.sync_copy(dst_hbm.at[wid], idx2d)
        pltpu.sync_copy(ones_hbm, obuf)
        pltpu.sync_copy(zer_hbm, acc_cnt.at[pl.ds(row0, N_PER_TILE)])
        plsc.subcore_barrier()

        def step(i, carry):
            @pl.when(i > 0)
            def _():
                pltpu.make_async_copy(obuf, acc_cnt.at[idx2d.at[0]],
                                      scnt).wait()

            pltpu.async_copy(obuf, acc_cnt.at[idx2d.at[i]], scnt, add=True)
            return carry

        lax.fori_loop(0, NCHUNK, step, 0)
        pltpu.make_async_copy(obuf, acc_cnt.at[idx2d.at[0]], scnt).wait()
        plsc.subcore_barrier()
        pltpu.sync_copy(acc_cnt.at[pl.ds(row0, N_PER_TILE)],
                        cnt_hbm.at[pl.ds(cid * NPAD + row0, N_PER_TILE)])

    return k(dst3, zer, ones)


def _sc_scatter(vals, dst3, zer):
    npair = NCHUNK // 2

    @functools.partial(
        pl.kernel,
        out_type=jax.ShapeDtypeStruct((NC * NPAD, W), jnp.float32),
        mesh=_sc_mesh(),
        scratch_types=[
            pltpu.VMEM((NCHUNK, CHUNK), jnp.int32),
            pltpu.VMEM((CHUNK, W), jnp.float32),
            pltpu.VMEM((CHUNK, W), jnp.float32),
            pltpu.VMEM_SHARED((NPAD, W), jnp.float32),
            pltpu.SemaphoreType.DMA,
            pltpu.SemaphoreType.DMA,
            pltpu.SemaphoreType.DMA,
            pltpu.SemaphoreType.DMA,
        ],
    )
    def k(vals_hbm, dst_hbm, zer_hbm, out_hbm,
          idx2d, vb0, vb1, acc, sl0, sl1, sc0, sc1):
        cid = lax.axis_index("c")
        tid = lax.axis_index("s")
        wid = tid * NC + cid
        base = wid * E_PER_W
        row0 = tid * N_PER_TILE
        pltpu.sync_copy(dst_hbm.at[wid], idx2d)
        pltpu.sync_copy(zer_hbm, acc.at[pl.ds(row0, N_PER_TILE)])
        plsc.subcore_barrier()
        slots = ((vb0, sl0, sc0), (vb1, sl1, sc1))

        def issue_load(ci, slot):
            vb, sl, _ = slots[slot]
            pltpu.async_copy(vals_hbm.at[pl.ds(base + ci * CHUNK, CHUNK)],
                             vb, sl)

        def drain_load(slot):
            vb, sl, _ = slots[slot]
            pltpu.make_async_copy(vals_hbm.at[pl.ds(base, CHUNK)],
                                  vb, sl).wait()

        def issue_scat(ci, slot):
            vb, _, sc = slots[slot]
            pltpu.async_copy(vb, acc.at[idx2d.at[ci]], sc, add=True)

        def drain_scat(slot):
            vb, _, sc = slots[slot]
            pltpu.make_async_copy(vb, acc.at[idx2d.at[0]], sc).wait()

        issue_load(0, 0)

        def pair(j, carry):
            a = 2 * j

            @pl.when(j > 0)
            def _():
                drain_scat(1)

            issue_load(a + 1, 1)
            drain_load(0)
            issue_scat(a, 0)

            drain_scat(0)
            issue_load(a + 2, 0)
            drain_load(1)
            issue_scat(a + 1, 1)
            return carry

        lax.fori_loop(0, npair, pair, 0)
        drain_scat(1)
        drain_load(0)
        issue_scat(NCHUNK - 1, 0)
        drain_scat(0)
        plsc.subcore_barrier()
        pltpu.sync_copy(acc.at[pl.ds(row0, N_PER_TILE)],
                        out_hbm.at[pl.ds(cid * NPAD + row0, N_PER_TILE)])

    return k(vals, dst3, zer)


def kernel(c, e, edge_index, batch, W_in_node, b_in_node, W_in_edge,
           b_in_edge, blocks):
    f32 = jnp.float32
    src3 = edge_index[0].reshape(NW, NCHUNK, CHUNK)
    dst3 = edge_index[1].reshape(NW, NCHUNK, CHUNK)

    def bc(b):
        return jnp.broadcast_to(b[None, :], (8, W)).astype(f32)

    zer = jnp.zeros((N_PER_TILE, W), f32)
    ones = jnp.ones((CHUNK, W), f32)

    p0, p1 = blocks
    a0, b0_, c0 = p0["We1"][:W], p0["We1"][W:2 * W], p0["We1"][2 * W:]
    a1, b1_, c1 = p1["We1"][:W], p1["We1"][W:2 * W], p1["We1"][2 * W:]
    n1a0, n1b0 = p0["Wn1"][:W], p0["Wn1"][W:]
    n1a1, n1b1 = p1["Wn1"][:W], p1["Wn1"][W:]

    cnt_flat = _sc_counts(dst3, zer, ones)
    cnt = cnt_flat.reshape(NC, NPAD, W)
    v0, pt0, qt0 = _node_init(c, W_in_node, bc(b_in_node), a0, b0_,
                              bc(p0["be1"]))
    gs0, gd0 = _sc_gather(pt0, qt0, src3, dst3)
    e1 = _edge0(gs0, gd0, e, W_in_edge, bc(b_in_edge), c0, p0["We2"],
                bc(p0["be2"]))
    s0_flat = _sc_scatter(e1, dst3, zer)
    s0 = s0_flat.reshape(NC, NPAD, W)
    v1, pt1, qt1 = _node_step(v0, s0, cnt, n1a0, n1b0, bc(p0["bn1"]),
                              p0["Wn2"], bc(p0["bn2"]), a1, b1_,
                              bc(p1["be1"]))
    gs1, gd1 = _sc_gather(pt1, qt1, src3, dst3)
    e2, ssum, ssq = _edge1(gs1, gd1, e1, c1, p1["We2"], bc(p1["be2"]))
    s1_flat = _sc_scatter(e2, dst3, zer)
    s1 = s1_flat.reshape(NC, NPAD, W)
    c_bn = _node_last(v1, s1, cnt, n1a1, n1b1, bc(p1["bn1"]), p1["Wn2"],
                      bc(p1["bn2"]))
    e_bn = _bn_apply(e2, ssum, ssq)
    return (c_bn, e_bn, edge_index, batch)


# speedup vs baseline: 1.0956x; 1.0083x over previous
import functools

import jax
import jax.numpy as jnp
from jax import lax
from jax.experimental import pallas as pl
from jax.experimental.pallas import tpu as pltpu
from jax.experimental.pallas import tpu_sc as plsc

N_NODES = 10000
E_EDGES = 320000
W = 128
EPS = 1e-5

NC = 2
NS = 16
NW = NC * NS
E_PER_W = E_EDGES // NW
CHUNK = 80
NCHUNK = E_PER_W // CHUNK
NPAD = 10240
N_PER_TILE = NPAD // NS

BE = 8000
NBE = E_EDGES // BE
BN_ROWS = 2000
NBN = N_NODES // BN_ROWS

_SELU_ALPHA = 1.6732632423543772848170429916717
_SELU_SCALE = 1.0507009873554804934193349852946


def _selu(x):
    return _SELU_SCALE * jnp.where(x > 0, x, _SELU_ALPHA * (jnp.exp(x) - 1.0))


def _dot(a, b):
    return jnp.dot(a, b, preferred_element_type=jnp.float32)


def _node_init_body(c_ref, wn_ref, bn_ref, a_ref, b_ref, be1_ref,
                    v_ref, p_ref, q_ref):
    v = _dot(c_ref[...], wn_ref[...]) + bn_ref[0:1, :]
    v_ref[...] = v
    p_ref[...] = _dot(v, a_ref[...]) + be1_ref[0:1, :]
    q_ref[...] = _dot(v, b_ref[...])


def _node_init(c, wn, bn, a, b, be1):
    out_sh = jax.ShapeDtypeStruct((N_NODES, W), jnp.float32)
    wspec = lambda sh: pl.BlockSpec(sh, lambda i: (0,) * len(sh))
    return pl.pallas_call(
        _node_init_body,
        grid=(NBN,),
        in_specs=[
            pl.BlockSpec((BN_ROWS, 4), lambda i: (i, 0)),
            wspec((4, W)), wspec((8, W)), wspec((W, W)), wspec((W, W)),
            wspec((8, W)),
        ],
        out_specs=[pl.BlockSpec((BN_ROWS, W), lambda i: (i, 0))] * 3,
        out_shape=[out_sh] * 3,
    )(c, wn, bn, a, b, be1)


def _edge0_body(gs_ref, gd_ref, er_ref, wie_ref, bie_ref, c0_ref, we2_ref,
                be2_ref, e1_ref):
    e0 = _dot(er_ref[...], wie_ref[...]) + bie_ref[0:1, :]
    h = _selu(gs_ref[...] + gd_ref[...] + _dot(e0, c0_ref[...]))
    e1_ref[...] = e0 + _dot(h, we2_ref[...]) + be2_ref[0:1, :]


def _edge0(gs, gd, e_raw, wie, bie, c0, we2, be2):
    wspec = lambda sh: pl.BlockSpec(sh, lambda i: (0,) * len(sh))
    espec = pl.BlockSpec((BE, W), lambda i: (i, 0))
    return pl.pallas_call(
        _edge0_body,
        grid=(NBE,),
        in_specs=[
            espec, espec,
            pl.BlockSpec((BE, 4), lambda i: (i, 0)),
            wspec((4, W)), wspec((8, W)), wspec((W, W)), wspec((W, W)),
            wspec((8, W)),
        ],
        out_specs=espec,
        out_shape=jax.ShapeDtypeStruct((E_EDGES, W), jnp.float32),
    )(gs, gd, e_raw, wie, bie, c0, we2, be2)


def _edge1_body(gs_ref, gd_ref, e1_ref, c1_ref, we2_ref, be2_ref,
                e2_ref, sum_ref, sq_ref):
    h = _selu(gs_ref[...] + gd_ref[...] + _dot(e1_ref[...], c1_ref[...]))
    e2 = e1_ref[...] + _dot(h, we2_ref[...]) + be2_ref[0:1, :]
    e2_ref[...] = e2

    @pl.when(pl.program_id(0) == 0)
    def _():
        sum_ref[...] = jnp.zeros_like(sum_ref)
        sq_ref[...] = jnp.zeros_like(sq_ref)

    ps = jnp.sum(e2, axis=0, keepdims=True)
    pq = jnp.sum(e2 * e2, axis=0, keepdims=True)
    sum_ref[...] += jnp.broadcast_to(ps, sum_ref.shape)
    sq_ref[...] += jnp.broadcast_to(pq, sq_ref.shape)


def _edge1(gs, gd, e1, c1, we2, be2):
    wspec = lambda sh: pl.BlockSpec(sh, lambda i: (0,) * len(sh))
    espec = pl.BlockSpec((BE, W), lambda i: (i, 0))
    return pl.pallas_call(
        _edge1_body,
        grid=(NBE,),
        in_specs=[espec, espec, espec, wspec((W, W)), wspec((W, W)),
                  wspec((8, W))],
        out_specs=[espec, wspec((8, W)), wspec((8, W))],
        out_shape=[jax.ShapeDtypeStruct((E_EDGES, W), jnp.float32),
                   jax.ShapeDtypeStruct((8, W), jnp.float32),
                   jax.ShapeDtypeStruct((8, W), jnp.float32)],
    )(gs, gd, e1, c1, we2, be2)


def _node_step_body(v_ref, s_ref, cnt_ref, n1a_ref, n1b_ref, bn1_ref,
                    wn2_ref, bn2_ref, a_ref, b_ref, be1_ref,
                    vn_ref, p_ref, q_ref):
    s = s_ref[0] + s_ref[1]
    cnt = cnt_ref[0] + cnt_ref[1]
    m = s * (1.0 / jnp.maximum(cnt[:, 0:1], 1.0))
    h = _selu(_dot(v_ref[...], n1a_ref[...]) + _dot(m, n1b_ref[...])
              + bn1_ref[0:1, :])
    vn = v_ref[...] + _dot(h, wn2_ref[...]) + bn2_ref[0:1, :]
    vn_ref[...] = vn
    p_ref[...] = _dot(vn, a_ref[...]) + be1_ref[0:1, :]
    q_ref[...] = _dot(vn, b_ref[...])


def _node_step(v, s_parts, cnt_parts, n1a, n1b, bn1, wn2, bn2, a, b, be1):
    wspec = lambda sh: pl.BlockSpec(sh, lambda i: (0,) * len(sh))
    nspec = pl.BlockSpec((BN_ROWS, W), lambda i: (i, 0))
    return pl.pallas_call(
        _node_step_body,
        grid=(NBN,),
        in_specs=[
            nspec,
            pl.BlockSpec((NC, BN_ROWS, W), lambda i: (0, i, 0)),
            pl.BlockSpec((NC, BN_ROWS, W), lambda i: (0, i, 0)),
            wspec((W, W)), wspec((W, W)), wspec((8, W)), wspec((W, W)),
            wspec((8, W)), wspec((W, W)), wspec((W, W)), wspec((8, W)),
        ],
        out_specs=[nspec] * 3,
        out_shape=[jax.ShapeDtypeStruct((N_NODES, W), jnp.float32)] * 3,
    )(v, s_parts, cnt_parts, n1a, n1b, bn1, wn2, bn2, a, b, be1)


def _node_last_body(v_ref, s_ref, cnt_ref, n1a_ref, n1b_ref, bn1_ref,
                    wn2_ref, bn2_ref, vn_ref):
    s = s_ref[0] + s_ref[1]
    cnt = cnt_ref[0] + cnt_ref[1]
    m = s * (1.0 / jnp.maximum(cnt[:, 0:1], 1.0))
    h = _selu(_dot(v_ref[...], n1a_ref[...]) + _dot(m, n1b_ref[...])
              + bn1_ref[0:1, :])
    vn = v_ref[...] + _dot(h, wn2_ref[...]) + bn2_ref[0:1, :]
    mu = jnp.mean(vn, axis=0, keepdims=True)
    var = jnp.mean((vn - mu) * (vn - mu), axis=0, keepdims=True)
    vn_ref[...] = (vn - mu) * lax.rsqrt(var + EPS)


def _node_last(v, s_parts, cnt_parts, n1a, n1b, bn1, wn2, bn2):
    wspec = lambda sh: pl.BlockSpec(sh, lambda i: (0,) * len(sh))
    nspec = pl.BlockSpec((N_NODES, W), lambda i: (0, 0))
    return pl.pallas_call(
        _node_last_body,
        grid=(1,),
        in_specs=[
            nspec,
            pl.BlockSpec((NC, N_NODES, W), lambda i: (0, 0, 0)),
            pl.BlockSpec((NC, N_NODES, W), lambda i: (0, 0, 0)),
            wspec((W, W)), wspec((W, W)), wspec((8, W)), wspec((W, W)),
            wspec((8, W)),
        ],
        out_specs=nspec,
        out_shape=jax.ShapeDtypeStruct((N_NODES, W), jnp.float32),
    )(v, s_parts, cnt_parts, n1a, n1b, bn1, wn2, bn2)


def _bn_apply_body(x_ref, sum_ref, sq_ref, o_ref):
    inv_n = 1.0 / E_EDGES
    mu = sum_ref[0:1, :] * inv_n
    var = sq_ref[0:1, :] * inv_n - mu * mu
    o_ref[...] = (x_ref[...] - mu) * lax.rsqrt(var + EPS)


def _bn_apply(x, ssum, ssq):
    wspec = lambda sh: pl.BlockSpec(sh, lambda i: (0,) * len(sh))
    espec = pl.BlockSpec((BE, W), lambda i: (i, 0))
    return pl.pallas_call(
        _bn_apply_body,
        grid=(NBE,),
        in_specs=[espec, wspec((8, W)), wspec((8, W))],
        out_specs=espec,
        out_shape=jax.ShapeDtypeStruct((E_EDGES, W), jnp.float32),
    )(x, ssum, ssq)


def _sc_mesh():
    return plsc.VectorSubcoreMesh(core_axis_name="c", subcore_axis_name="s")


def _sc_gather(p, q, src3, dst3):
    CH2 = 2 * CHUNK
    nsc = NCHUNK // 2
    npair = nsc // 2

    @functools.partial(
        pl.kernel,
        out_type=(jax.ShapeDtypeStruct((E_EDGES, W), jnp.float32),
                  jax.ShapeDtypeStruct((E_EDGES, W), jnp.float32)),
        mesh=_sc_mesh(),
        scratch_types=[
            pltpu.VMEM((NCHUNK, CHUNK), jnp.int32),
            pltpu.VMEM((NCHUNK, CHUNK), jnp.int32),
            pltpu.VMEM((CH2, W), jnp.float32),
            pltpu.VMEM((CH2, W), jnp.float32),
            pltpu.VMEM((CH2, W), jnp.float32),
            pltpu.VMEM((CH2, W), jnp.float32),
            pltpu.SemaphoreType.DMA,
            pltpu.SemaphoreType.DMA,
            pltpu.SemaphoreType.DMA,
            pltpu.SemaphoreType.DMA,
        ],
    )
    def k(p_hbm, q_hbm, src_hbm, dst_hbm, gs_hbm, gd_hbm,
          idx_s, idx_d, bs0, bd0, bs1, bd1, sg0, sg1, st0, st1):
        wid = lax.axis_index("s") * NC + lax.axis_index("c")
        base = wid * E_PER_W
        pltpu.sync_copy(src_hbm.at[wid], idx_s)
        pltpu.sync_copy(dst_hbm.at[wid], idx_d)
        slots = ((bs0, bd0, sg0, st0), (bs1, bd1, sg1, st1))
        lo = pl.ds(0, CHUNK)
        hi = pl.ds(CHUNK, CHUNK)

        def issue_gather(si, slot):
            bs, bd, sg, _ = slots[slot]
            c0 = 2 * si
            pltpu.async_copy(p_hbm.at[idx_s.at[c0]], bs.at[lo], sg)
            pltpu.async_copy(p_hbm.at[idx_s.at[c0 + 1]], bs.at[hi], sg)
            pltpu.async_copy(q_hbm.at[idx_d.at[c0]], bd.at[lo], sg)
            pltpu.async_copy(q_hbm.at[idx_d.at[c0 + 1]], bd.at[hi], sg)

        def drain_gather(slot):
            bs, bd, sg, _ = slots[slot]
            for buf in (bs, bd):
                pltpu.make_async_copy(p_hbm.at[idx_s.at[0]],
                                      buf.at[lo], sg).wait()
                pltpu.make_async_copy(p_hbm.at[idx_s.at[0]],
                                      buf.at[hi], sg).wait()

        def issue_store(si, slot):
            bs, bd, _, st = slots[slot]
            rows = pl.ds(base + si * CH2, CH2)
            pltpu.async_copy(bs, gs_hbm.at[rows], st)
            pltpu.async_copy(bd, gd_hbm.at[rows], st)

        def drain_store(slot):
            bs, bd, _, st = slots[slot]
            rows = pl.ds(base, CH2)
            pltpu.make_async_copy(bs, gs_hbm.at[rows], st).wait()
            pltpu.make_async_copy(bd, gd_hbm.at[rows], st).wait()

        issue_gather(0, 0)

        def pair(j, carry):
            a = 2 * j

            @pl.when(j > 0)
            def _():
                drain_store(1)

            issue_gather(a + 1, 1)
            drain_gather(0)
            issue_store(a, 0)

            drain_store(0)

            @pl.when(a + 2 < nsc)
            def _():
                issue_gather(a + 2, 0)

            drain_gather(1)
            issue_store(a + 1, 1)
            return carry

        lax.fori_loop(0, npair, pair, 0)
        drain_store(1)
        last = NCHUNK - 1
        pltpu.async_copy(p_hbm.at[idx_s.at[last]], bs0.at[lo], sg0)
        pltpu.async_copy(q_hbm.at[idx_d.at[last]], bd0.at[lo], sg0)
        pltpu.make_async_copy(p_hbm.at[idx_s.at[0]], bs0.at[lo], sg0).wait()
        pltpu.make_async_copy(p_hbm.at[idx_s.at[0]], bd0.at[lo], sg0).wait()
        rows = pl.ds(base + last * CHUNK, CHUNK)
        pltpu.async_copy(bs0.at[lo], gs_hbm.at[rows], st0)
        pltpu.async_copy(bd0.at[lo], gd_hbm.at[rows], st0)
        pltpu.make_async_copy(bs0.at[lo], gs_hbm.at[rows], st0).wait()
        pltpu.make_async_copy(bd0.at[lo], gd_hbm.at[rows], st0).wait()

    return k(p, q, src3, dst3)


def _sc_counts(dst3, zer, ones):
    @functools.partial(
        pl.kernel,
        out_type=jax.ShapeDtypeStruct((NC * NPAD, W), jnp.float32),
        mesh=_sc_mesh(),
        scratch_types=[
            pltpu.VMEM((NCHUNK, CHUNK), jnp.int32),
            pltpu.VMEM((CHUNK, W), jnp.float32),
            pltpu.VMEM_SHARED((NPAD, W), jnp.float32),
            pltpu.SemaphoreType.DMA,
        ],
    )
    def k(dst_hbm, zer_hbm, ones_hbm, cnt_hbm, idx2d, obuf, acc_cnt, scnt):
        cid = lax.axis_index("c")
        tid = lax.axis_index("s")
        wid = tid * NC + cid
        row0 = tid * N_PER_TILE
        pltpu.sync_copy(dst_hbm.at[wid], idx2d)
        pltpu.sync_copy(ones_hbm, obuf)
        pltpu.sync_copy(zer_hbm, acc_cnt.at[pl.ds(row0, N_PER_TILE)])
        plsc.subcore_barrier()

        def step(i, carry):
            @pl.when(i > 0)
            def _():
                pltpu.make_async_copy(obuf, acc_cnt.at[idx2d.at[0]],
                                      scnt).wait()

            pltpu.async_copy(obuf, acc_cnt.at[idx2d.at[i]], scnt, add=True)
            return carry

        lax.fori_loop(0, NCHUNK, step, 0)
        pltpu.make_async_copy(obuf, acc_cnt.at[idx2d.at[0]], scnt).wait()
        plsc.subcore_barrier()
        pltpu.sync_copy(acc_cnt.at[pl.ds(row0, N_PER_TILE)],
                        cnt_hbm.at[pl.ds(cid * NPAD + row0, N_PER_TILE)])

    return k(dst3, zer, ones)


def _sc_scatter(vals, dst3, zer):
    npair = NCHUNK // 2

    @functools.partial(
        pl.kernel,
        out_type=jax.ShapeDtypeStruct((NC * NPAD, W), jnp.float32),
        mesh=_sc_mesh(),
        scratch_types=[
            pltpu.VMEM((NCHUNK, CHUNK), jnp.int32),
            pltpu.VMEM((CHUNK, W), jnp.float32),
            pltpu.VMEM((CHUNK, W), jnp.float32),
            pltpu.VMEM_SHARED((NPAD, W), jnp.float32),
            pltpu.SemaphoreType.DMA,
            pltpu.SemaphoreType.DMA,
            pltpu.SemaphoreType.DMA,
            pltpu.SemaphoreType.DMA,
        ],
    )
    def k(vals_hbm, dst_hbm, zer_hbm, out_hbm,
          idx2d, vb0, vb1, acc, sl0, sl1, sc0, sc1):
        cid = lax.axis_index("c")
        tid = lax.axis_index("s")
        wid = tid * NC + cid
        base = wid * E_PER_W
        row0 = tid * N_PER_TILE
        pltpu.sync_copy(dst_hbm.at[wid], idx2d)
        pltpu.sync_copy(zer_hbm, acc.at[pl.ds(row0, N_PER_TILE)])
        plsc.subcore_barrier()
        slots = ((vb0, sl0, sc0), (vb1, sl1, sc1))

        def issue_load(ci, slot):
            vb, sl, _ = slots[slot]
            pltpu.async_copy(vals_hbm.at[pl.ds(base + ci * CHUNK, CHUNK)],
                             vb, sl)

        def drain_load(slot):
            vb, sl, _ = slots[slot]
            pltpu.make_async_copy(vals_hbm.at[pl.ds(base, CHUNK)],
                                  vb, sl).wait()

        def issue_scat(ci, slot):
            vb, _, sc = slots[slot]
            pltpu.async_copy(vb, acc.at[idx2d.at[ci]], sc, add=True)

        def drain_scat(slot):
            vb, _, sc = slots[slot]
            pltpu.make_async_copy(vb, acc.at[idx2d.at[0]], sc).wait()

        issue_load(0, 0)

        def pair(j, carry):
            a = 2 * j

            @pl.when(j > 0)
            def _():
                drain_scat(1)

            issue_load(a + 1, 1)
            drain_load(0)
            issue_scat(a, 0)

            drain_scat(0)
            issue_load(a + 2, 0)
            drain_load(1)
            issue_scat(a + 1, 1)
            return carry

        lax.fori_loop(0, npair, pair, 0)
        drain_scat(1)
        drain_load(0)
        issue_scat(NCHUNK - 1, 0)
        drain_scat(0)
        plsc.subcore_barrier()
        pltpu.sync_copy(acc.at[pl.ds(row0, N_PER_TILE)],
                        out_hbm.at[pl.ds(cid * NPAD + row0, N_PER_TILE)])

    return k(vals, dst3, zer)


def kernel(c, e, edge_index, batch, W_in_node, b_in_node, W_in_edge,
           b_in_edge, blocks):
    f32 = jnp.float32
    src3 = edge_index[0].reshape(NW, NCHUNK, CHUNK)
    dst3 = edge_index[1].reshape(NW, NCHUNK, CHUNK)

    def bc(b):
        return jnp.broadcast_to(b[None, :], (8, W)).astype(f32)

    zer = jnp.zeros((N_PER_TILE, W), f32)
    ones = jnp.ones((CHUNK, W), f32)

    p0, p1 = blocks
    a0, b0_, c0 = p0["We1"][:W], p0["We1"][W:2 * W], p0["We1"][2 * W:]
    a1, b1_, c1 = p1["We1"][:W], p1["We1"][W:2 * W], p1["We1"][2 * W:]
    n1a0, n1b0 = p0["Wn1"][:W], p0["Wn1"][W:]
    n1a1, n1b1 = p1["Wn1"][:W], p1["Wn1"][W:]

    cnt_flat = _sc_counts(dst3, zer, ones)
    cnt = cnt_flat.reshape(NC, NPAD, W)
    v0, pt0, qt0 = _node_init(c, W_in_node, bc(b_in_node), a0, b0_,
                              bc(p0["be1"]))
    gs0, gd0 = _sc_gather(pt0, qt0, src3, dst3)
    e1 = _edge0(gs0, gd0, e, W_in_edge, bc(b_in_edge), c0, p0["We2"],
                bc(p0["be2"]))
    s0_flat = _sc_scatter(e1, dst3, zer)
    s0 = s0_flat.reshape(NC, NPAD, W)
    v1, pt1, qt1 = _node_step(v0, s0, cnt, n1a0, n1b0, bc(p0["bn1"]),
                              p0["Wn2"], bc(p0["bn2"]), a1, b1_,
                              bc(p1["be1"]))
    gs1, gd1 = _sc_gather(pt1, qt1, src3, dst3)
    e2, ssum, ssq = _edge1(gs1, gd1, e1, c1, p1["We2"], bc(p1["be2"]))
    s1_flat = _sc_scatter(e2, dst3, zer)
    s1 = s1_flat.reshape(NC, NPAD, W)
    c_bn = _node_last(v1, s1, cnt, n1a1, n1b1, bc(p1["bn1"]), p1["Wn2"],
                      bc(p1["bn2"]))
    e_bn = _bn_apply(e2, ssum, ssq)
    return (c_bn, e_bn, edge_index, batch)


# speedup vs baseline: 1.0974x; 1.0016x over previous
import functools

import jax
import jax.numpy as jnp
from jax import lax
from jax.experimental import pallas as pl
from jax.experimental.pallas import tpu as pltpu
from jax.experimental.pallas import tpu_sc as plsc

N_NODES = 10000
E_EDGES = 320000
W = 128
EPS = 1e-5

NC = 2
NS = 16
NW = NC * NS
E_PER_W = E_EDGES // NW
CHUNK = 80
NCHUNK = E_PER_W // CHUNK
NPAD = 10240
N_PER_TILE = NPAD // NS

BE = 10000
NBE = E_EDGES // BE
BN_ROWS = 2000
NBN = N_NODES // BN_ROWS

_SELU_ALPHA = 1.6732632423543772848170429916717
_SELU_SCALE = 1.0507009873554804934193349852946


def _selu(x):
    return _SELU_SCALE * jnp.where(x > 0, x, _SELU_ALPHA * (jnp.exp(x) - 1.0))


def _dot(a, b):
    return jnp.dot(a, b, preferred_element_type=jnp.float32)


def _node_init_body(c_ref, wn_ref, bn_ref, a_ref, b_ref, be1_ref,
                    v_ref, p_ref, q_ref):
    v = _dot(c_ref[...], wn_ref[...]) + bn_ref[0:1, :]
    v_ref[...] = v
    p_ref[...] = _dot(v, a_ref[...]) + be1_ref[0:1, :]
    q_ref[...] = _dot(v, b_ref[...])


def _node_init(c, wn, bn, a, b, be1):
    out_sh = jax.ShapeDtypeStruct((N_NODES, W), jnp.float32)
    wspec = lambda sh: pl.BlockSpec(sh, lambda i: (0,) * len(sh))
    return pl.pallas_call(
        _node_init_body,
        grid=(NBN,),
        in_specs=[
            pl.BlockSpec((BN_ROWS, 4), lambda i: (i, 0)),
            wspec((4, W)), wspec((8, W)), wspec((W, W)), wspec((W, W)),
            wspec((8, W)),
        ],
        out_specs=[pl.BlockSpec((BN_ROWS, W), lambda i: (i, 0))] * 3,
        out_shape=[out_sh] * 3,
    )(c, wn, bn, a, b, be1)


def _edge0_body(gs_ref, gd_ref, er_ref, wie_ref, bie_ref, c0_ref, we2_ref,
                be2_ref, e1_ref):
    e0 = _dot(er_ref[...], wie_ref[...]) + bie_ref[0:1, :]
    h = _selu(gs_ref[...] + gd_ref[...] + _dot(e0, c0_ref[...]))
    e1_ref[...] = e0 + _dot(h, we2_ref[...]) + be2_ref[0:1, :]


def _edge0(gs, gd, e_raw, wie, bie, c0, we2, be2):
    wspec = lambda sh: pl.BlockSpec(sh, lambda i: (0,) * len(sh))
    espec = pl.BlockSpec((BE, W), lambda i: (i, 0))
    return pl.pallas_call(
        _edge0_body,
        grid=(NBE,),
        in_specs=[
            espec, espec,
            pl.BlockSpec((BE, 4), lambda i: (i, 0)),
            wspec((4, W)), wspec((8, W)), wspec((W, W)), wspec((W, W)),
            wspec((8, W)),
        ],
        out_specs=espec,
        out_shape=jax.ShapeDtypeStruct((E_EDGES, W), jnp.float32),
    )(gs, gd, e_raw, wie, bie, c0, we2, be2)


def _edge1_body(gs_ref, gd_ref, e1_ref, c1_ref, we2_ref, be2_ref,
                e2_ref, sum_ref, sq_ref):
    h = _selu(gs_ref[...] + gd_ref[...] + _dot(e1_ref[...], c1_ref[...]))
    e2 = e1_ref[...] + _dot(h, we2_ref[...]) + be2_ref[0:1, :]
    e2_ref[...] = e2

    @pl.when(pl.program_id(0) == 0)
    def _():
        sum_ref[...] = jnp.zeros_like(sum_ref)
        sq_ref[...] = jnp.zeros_like(sq_ref)

    ps = jnp.sum(e2, axis=0, keepdims=True)
    pq = jnp.sum(e2 * e2, axis=0, keepdims=True)
    sum_ref[...] += jnp.broadcast_to(ps, sum_ref.shape)
    sq_ref[...] += jnp.broadcast_to(pq, sq_ref.shape)


def _edge1(gs, gd, e1, c1, we2, be2):
    wspec = lambda sh: pl.BlockSpec(sh, lambda i: (0,) * len(sh))
    espec = pl.BlockSpec((BE, W), lambda i: (i, 0))
    return pl.pallas_call(
        _edge1_body,
        grid=(NBE,),
        in_specs=[espec, espec, espec, wspec((W, W)), wspec((W, W)),
                  wspec((8, W))],
        out_specs=[espec, wspec((8, W)), wspec((8, W))],
        out_shape=[jax.ShapeDtypeStruct((E_EDGES, W), jnp.float32),
                   jax.ShapeDtypeStruct((8, W), jnp.float32),
                   jax.ShapeDtypeStruct((8, W), jnp.float32)],
    )(gs, gd, e1, c1, we2, be2)


def _node_step_body(v_ref, s_ref, cnt_ref, n1a_ref, n1b_ref, bn1_ref,
                    wn2_ref, bn2_ref, a_ref, b_ref, be1_ref,
                    vn_ref, p_ref, q_ref):
    s = s_ref[0] + s_ref[1]
    cnt = cnt_ref[0] + cnt_ref[1]
    m = s * (1.0 / jnp.maximum(cnt[:, 0:1], 1.0))
    h = _selu(_dot(v_ref[...], n1a_ref[...]) + _dot(m, n1b_ref[...])
              + bn1_ref[0:1, :])
    vn = v_ref[...] + _dot(h, wn2_ref[...]) + bn2_ref[0:1, :]
    vn_ref[...] = vn
    p_ref[...] = _dot(vn, a_ref[...]) + be1_ref[0:1, :]
    q_ref[...] = _dot(vn, b_ref[...])


def _node_step(v, s_parts, cnt_parts, n1a, n1b, bn1, wn2, bn2, a, b, be1):
    wspec = lambda sh: pl.BlockSpec(sh, lambda i: (0,) * len(sh))
    nspec = pl.BlockSpec((BN_ROWS, W), lambda i: (i, 0))
    return pl.pallas_call(
        _node_step_body,
        grid=(NBN,),
        in_specs=[
            nspec,
            pl.BlockSpec((NC, BN_ROWS, W), lambda i: (0, i, 0)),
            pl.BlockSpec((NC, BN_ROWS, W), lambda i: (0, i, 0)),
            wspec((W, W)), wspec((W, W)), wspec((8, W)), wspec((W, W)),
            wspec((8, W)), wspec((W, W)), wspec((W, W)), wspec((8, W)),
        ],
        out_specs=[nspec] * 3,
        out_shape=[jax.ShapeDtypeStruct((N_NODES, W), jnp.float32)] * 3,
    )(v, s_parts, cnt_parts, n1a, n1b, bn1, wn2, bn2, a, b, be1)


def _node_last_body(v_ref, s_ref, cnt_ref, n1a_ref, n1b_ref, bn1_ref,
                    wn2_ref, bn2_ref, vn_ref):
    s = s_ref[0] + s_ref[1]
    cnt = cnt_ref[0] + cnt_ref[1]
    m = s * (1.0 / jnp.maximum(cnt[:, 0:1], 1.0))
    h = _selu(_dot(v_ref[...], n1a_ref[...]) + _dot(m, n1b_ref[...])
              + bn1_ref[0:1, :])
    vn = v_ref[...] + _dot(h, wn2_ref[...]) + bn2_ref[0:1, :]
    mu = jnp.mean(vn, axis=0, keepdims=True)
    var = jnp.mean((vn - mu) * (vn - mu), axis=0, keepdims=True)
    vn_ref[...] = (vn - mu) * lax.rsqrt(var + EPS)


def _node_last(v, s_parts, cnt_parts, n1a, n1b, bn1, wn2, bn2):
    wspec = lambda sh: pl.BlockSpec(sh, lambda i: (0,) * len(sh))
    nspec = pl.BlockSpec((N_NODES, W), lambda i: (0, 0))
    return pl.pallas_call(
        _node_last_body,
        grid=(1,),
        in_specs=[
            nspec,
            pl.BlockSpec((NC, N_NODES, W), lambda i: (0, 0, 0)),
            pl.BlockSpec((NC, N_NODES, W), lambda i: (0, 0, 0)),
            wspec((W, W)), wspec((W, W)), wspec((8, W)), wspec((W, W)),
            wspec((8, W)),
        ],
        out_specs=nspec,
        out_shape=jax.ShapeDtypeStruct((N_NODES, W), jnp.float32),
    )(v, s_parts, cnt_parts, n1a, n1b, bn1, wn2, bn2)


def _bn_apply_body(x_ref, sum_ref, sq_ref, o_ref):
    inv_n = 1.0 / E_EDGES
    mu = sum_ref[0:1, :] * inv_n
    var = sq_ref[0:1, :] * inv_n - mu * mu
    o_ref[...] = (x_ref[...] - mu) * lax.rsqrt(var + EPS)


def _bn_apply(x, ssum, ssq):
    wspec = lambda sh: pl.BlockSpec(sh, lambda i: (0,) * len(sh))
    espec = pl.BlockSpec((BE, W), lambda i: (i, 0))
    return pl.pallas_call(
        _bn_apply_body,
        grid=(NBE,),
        in_specs=[espec, wspec((8, W)), wspec((8, W))],
        out_specs=espec,
        out_shape=jax.ShapeDtypeStruct((E_EDGES, W), jnp.float32),
    )(x, ssum, ssq)


def _sc_mesh():
    return plsc.VectorSubcoreMesh(core_axis_name="c", subcore_axis_name="s")


def _sc_gather(p, q, src3, dst3):
    CH2 = 2 * CHUNK
    nsc = NCHUNK // 2
    npair = nsc // 2

    @functools.partial(
        pl.kernel,
        out_type=(jax.ShapeDtypeStruct((E_EDGES, W), jnp.float32),
                  jax.ShapeDtypeStruct((E_EDGES, W), jnp.float32)),
        mesh=_sc_mesh(),
        scratch_types=[
            pltpu.VMEM((NCHUNK, CHUNK), jnp.int32),
            pltpu.VMEM((NCHUNK, CHUNK), jnp.int32),
            pltpu.VMEM((CH2, W), jnp.float32),
            pltpu.VMEM((CH2, W), jnp.float32),
            pltpu.VMEM((CH2, W), jnp.float32),
            pltpu.VMEM((CH2, W), jnp.float32),
            pltpu.SemaphoreType.DMA,
            pltpu.SemaphoreType.DMA,
            pltpu.SemaphoreType.DMA,
            pltpu.SemaphoreType.DMA,
        ],
    )
    def k(p_hbm, q_hbm, src_hbm, dst_hbm, gs_hbm, gd_hbm,
          idx_s, idx_d, bs0, bd0, bs1, bd1, sg0, sg1, st0, st1):
        wid = lax.axis_index("s") * NC + lax.axis_index("c")
        base = wid * E_PER_W
        pltpu.sync_copy(src_hbm.at[wid], idx_s)
        pltpu.sync_copy(dst_hbm.at[wid], idx_d)
        slots = ((bs0, bd0, sg0, st0), (bs1, bd1, sg1, st1))
        lo = pl.ds(0, CHUNK)
        hi = pl.ds(CHUNK, CHUNK)

        def issue_gather(si, slot):
            bs, bd, sg, _ = slots[slot]
            c0 = 2 * si
            pltpu.async_copy(p_hbm.at[idx_s.at[c0]], bs.at[lo], sg)
            pltpu.async_copy(p_hbm.at[idx_s.at[c0 + 1]], bs.at[hi], sg)
            pltpu.async_copy(q_hbm.at[idx_d.at[c0]], bd.at[lo], sg)
            pltpu.async_copy(q_hbm.at[idx_d.at[c0 + 1]], bd.at[hi], sg)

        def drain_gather(slot):
            bs, bd, sg, _ = slots[slot]
            for buf in (bs, bd):
                pltpu.make_async_copy(p_hbm.at[idx_s.at[0]],
                                      buf.at[lo], sg).wait()
                pltpu.make_async_copy(p_hbm.at[idx_s.at[0]],
                                      buf.at[hi], sg).wait()

        def issue_store(si, slot):
            bs, bd, _, st = slots[slot]
            rows = pl.ds(base + si * CH2, CH2)
            pltpu.async_copy(bs, gs_hbm.at[rows], st)
            pltpu.async_copy(bd, gd_hbm.at[rows], st)

        def drain_store(slot):
            bs, bd, _, st = slots[slot]
            rows = pl.ds(base, CH2)
            pltpu.make_async_copy(bs, gs_hbm.at[rows], st).wait()
            pltpu.make_async_copy(bd, gd_hbm.at[rows], st).wait()

        issue_gather(0, 0)

        def pair(j, carry):
            a = 2 * j

            @pl.when(j > 0)
            def _():
                drain_store(1)

            issue_gather(a + 1, 1)
            drain_gather(0)
            issue_store(a, 0)

            drain_store(0)

            @pl.when(a + 2 < nsc)
            def _():
                issue_gather(a + 2, 0)

            drain_gather(1)
            issue_store(a + 1, 1)
            return carry

        lax.fori_loop(0, npair, pair, 0)
        drain_store(1)
        last = NCHUNK - 1
        pltpu.async_copy(p_hbm.at[idx_s.at[last]], bs0.at[lo], sg0)
        pltpu.async_copy(q_hbm.at[idx_d.at[last]], bd0.at[lo], sg0)
        pltpu.make_async_copy(p_hbm.at[idx_s.at[0]], bs0.at[lo], sg0).wait()
        pltpu.make_async_copy(p_hbm.at[idx_s.at[0]], bd0.at[lo], sg0).wait()
        rows = pl.ds(base + last * CHUNK, CHUNK)
        pltpu.async_copy(bs0.at[lo], gs_hbm.at[rows], st0)
        pltpu.async_copy(bd0.at[lo], gd_hbm.at[rows], st0)
        pltpu.make_async_copy(bs0.at[lo], gs_hbm.at[rows], st0).wait()
        pltpu.make_async_copy(bd0.at[lo], gd_hbm.at[rows], st0).wait()

    return k(p, q, src3, dst3)


def _sc_counts(dst3, zer, ones):
    @functools.partial(
        pl.kernel,
        out_type=jax.ShapeDtypeStruct((NC * NPAD, W), jnp.float32),
        mesh=_sc_mesh(),
        scratch_types=[
            pltpu.VMEM((NCHUNK, CHUNK), jnp.int32),
            pltpu.VMEM((CHUNK, W), jnp.float32),
            pltpu.VMEM_SHARED((NPAD, W), jnp.float32),
            pltpu.SemaphoreType.DMA,
        ],
    )
    def k(dst_hbm, zer_hbm, ones_hbm, cnt_hbm, idx2d, obuf, acc_cnt, scnt):
        cid = lax.axis_index("c")
        tid = lax.axis_index("s")
        wid = tid * NC + cid
        row0 = tid * N_PER_TILE
        pltpu.sync_copy(dst_hbm.at[wid], idx2d)
        pltpu.sync_copy(ones_hbm, obuf)
        pltpu.sync_copy(zer_hbm, acc_cnt.at[pl.ds(row0, N_PER_TILE)])
        plsc.subcore_barrier()

        def step(i, carry):
            @pl.when(i > 0)
            def _():
                pltpu.make_async_copy(obuf, acc_cnt.at[idx2d.at[0]],
                                      scnt).wait()

            pltpu.async_copy(obuf, acc_cnt.at[idx2d.at[i]], scnt, add=True)
            return carry

        lax.fori_loop(0, NCHUNK, step, 0)
        pltpu.make_async_copy(obuf, acc_cnt.at[idx2d.at[0]], scnt).wait()
        plsc.subcore_barrier()
        pltpu.sync_copy(acc_cnt.at[pl.ds(row0, N_PER_TILE)],
                        cnt_hbm.at[pl.ds(cid * NPAD + row0, N_PER_TILE)])

    return k(dst3, zer, ones)


def _sc_scatter(vals, dst3, zer):
    npair = NCHUNK // 2

    @functools.partial(
        pl.kernel,
        out_type=jax.ShapeDtypeStruct((NC * NPAD, W), jnp.float32),
        mesh=_sc_mesh(),
        scratch_types=[
            pltpu.VMEM((NCHUNK, CHUNK), jnp.int32),
            pltpu.VMEM((CHUNK, W), jnp.float32),
            pltpu.VMEM((CHUNK, W), jnp.float32),
            pltpu.VMEM_SHARED((NPAD, W), jnp.float32),
            pltpu.SemaphoreType.DMA,
            pltpu.SemaphoreType.DMA,
            pltpu.SemaphoreType.DMA,
            pltpu.SemaphoreType.DMA,
        ],
    )
    def k(vals_hbm, dst_hbm, zer_hbm, out_hbm,
          idx2d, vb0, vb1, acc, sl0, sl1, sc0, sc1):
        cid = lax.axis_index("c")
        tid = lax.axis_index("s")
        wid = tid * NC + cid
        base = wid * E_PER_W
        row0 = tid * N_PER_TILE
        pltpu.sync_copy(dst_hbm.at[wid], idx2d)
        pltpu.sync_copy(zer_hbm, acc.at[pl.ds(row0, N_PER_TILE)])
        plsc.subcore_barrier()
        slots = ((vb0, sl0, sc0), (vb1, sl1, sc1))

        def issue_load(ci, slot):
            vb, sl, _ = slots[slot]
            pltpu.async_copy(vals_hbm.at[pl.ds(base + ci * CHUNK, CHUNK)],
                             vb, sl)

        def drain_load(slot):
            vb, sl, _ = slots[slot]
            pltpu.make_async_copy(vals_hbm.at[pl.ds(base, CHUNK)],
                                  vb, sl).wait()

        def issue_scat(ci, slot):
            vb, _, sc = slots[slot]
            pltpu.async_copy(vb, acc.at[idx2d.at[ci]], sc, add=True)

        def drain_scat(slot):
            vb, _, sc = slots[slot]
            pltpu.make_async_copy(vb, acc.at[idx2d.at[0]], sc).wait()

        issue_load(0, 0)

        def pair(j, carry):
            a = 2 * j

            @pl.when(j > 0)
            def _():
                drain_scat(1)

            issue_load(a + 1, 1)
            drain_load(0)
            issue_scat(a, 0)

            drain_scat(0)
            issue_load(a + 2, 0)
            drain_load(1)
            issue_scat(a + 1, 1)
            return carry

        lax.fori_loop(0, npair, pair, 0)
        drain_scat(1)
        drain_load(0)
        issue_scat(NCHUNK - 1, 0)
        drain_scat(0)
        plsc.subcore_barrier()
        pltpu.sync_copy(acc.at[pl.ds(row0, N_PER_TILE)],
                        out_hbm.at[pl.ds(cid * NPAD + row0, N_PER_TILE)])

    return k(vals, dst3, zer)


def kernel(c, e, edge_index, batch, W_in_node, b_in_node, W_in_edge,
           b_in_edge, blocks):
    f32 = jnp.float32
    src3 = edge_index[0].reshape(NW, NCHUNK, CHUNK)
    dst3 = edge_index[1].reshape(NW, NCHUNK, CHUNK)

    def bc(b):
        return jnp.broadcast_to(b[None, :], (8, W)).astype(f32)

    zer = jnp.zeros((N_PER_TILE, W), f32)
    ones = jnp.ones((CHUNK, W), f32)

    p0, p1 = blocks
    a0, b0_, c0 = p0["We1"][:W], p0["We1"][W:2 * W], p0["We1"][2 * W:]
    a1, b1_, c1 = p1["We1"][:W], p1["We1"][W:2 * W], p1["We1"][2 * W:]
    n1a0, n1b0 = p0["Wn1"][:W], p0["Wn1"][W:]
    n1a1, n1b1 = p1["Wn1"][:W], p1["Wn1"][W:]

    cnt_flat = _sc_counts(dst3, zer, ones)
    cnt = cnt_flat.reshape(NC, NPAD, W)
    v0, pt0, qt0 = _node_init(c, W_in_node, bc(b_in_node), a0, b0_,
                              bc(p0["be1"]))
    gs0, gd0 = _sc_gather(pt0, qt0, src3, dst3)
    e1 = _edge0(gs0, gd0, e, W_in_edge, bc(b_in_edge), c0, p0["We2"],
                bc(p0["be2"]))
    s0_flat = _sc_scatter(e1, dst3, zer)
    s0 = s0_flat.reshape(NC, NPAD, W)
    v1, pt1, qt1 = _node_step(v0, s0, cnt, n1a0, n1b0, bc(p0["bn1"]),
                              p0["Wn2"], bc(p0["bn2"]), a1, b1_,
                              bc(p1["be1"]))
    gs1, gd1 = _sc_gather(pt1, qt1, src3, dst3)
    e2, ssum, ssq = _edge1(gs1, gd1, e1, c1, p1["We2"], bc(p1["be2"]))
    s1_flat = _sc_scatter(e2, dst3, zer)
    s1 = s1_flat.reshape(NC, NPAD, W)
    c_bn = _node_last(v1, s1, cnt, n1a1, n1b1, bc(p1["bn1"]), p1["Wn2"],
                      bc(p1["bn2"]))
    e_bn = _bn_apply(e2, ssum, ssq)
    return (c_bn, e_bn, edge_index, batch)
